# baseline (speedup 1.0000x reference)
import functools

import jax
import jax.numpy as jnp
from jax import lax
from jax.experimental import pallas as pl
from jax.experimental.pallas import tpu as pltpu
from jax.experimental.pallas import tpu_sc as plsc

N = 10000
E = 320000
D = 128
H = 2
C = 64
HC = H * C
TD = 100

NPAD = 10240
NBLK = 1024
EBLK = 512

_SC_INFO = plsc.get_sparse_core_info()
NC = _SC_INFO.num_cores
NS = _SC_INFO.num_subcores
NW = NC * NS
EPW = E // NW
GW = 80
NWIN = EPW // GW
NPT = NPAD // NS


def _qkvg_body(x_ref, w_ref, b_ref, o_ref):
    o_ref[...] = (
        jnp.dot(x_ref[...], w_ref[...], preferred_element_type=jnp.float32)
        + b_ref[...]
    )


def _alpha_body(qd_ref, kvs_ref, t_ref, wt_ref, bt_ref, we_ref,
                ex0_ref, ex1_ref, wv_ref):
    enc = jnp.cos(t_ref[...] * wt_ref[...] + bt_ref[...])
    eb = jnp.dot(enc, we_ref[...], preferred_element_type=jnp.float32)
    a = qd_ref[...] * (kvs_ref[:, :HC] + eb)
    h0 = jnp.sum(a[:, :C], axis=1) * (1.0 / 8.0)
    h1 = jnp.sum(a[:, C:], axis=1) * (1.0 / 8.0)
    ex0 = jnp.exp(h0)
    ex1 = jnp.exp(h1)
    ex0_ref[...] = ex0[None, :]
    ex1_ref[...] = ex1[None, :]
    w = jnp.concatenate(
        [jnp.broadcast_to(ex0[:, None], (EBLK, C)),
         jnp.broadcast_to(ex1[:, None], (EBLK, C))], axis=1)
    wv_ref[...] = (kvs_ref[:, HC:] + eb) * w


def _final_body(p_ref, den_ref, skip_ref, o_ref):
    acc = p_ref[0] + p_ref[1]
    d0 = den_ref[0, 0] + den_ref[1, 0]
    d1 = den_ref[0, 1] + den_ref[1, 1]
    r0 = 1.0 / (d0 + 1e-16)
    r1 = 1.0 / (d1 + 1e-16)
    r = jnp.concatenate(
        [jnp.broadcast_to(r0[:, None], (NBLK, C)),
         jnp.broadcast_to(r1[:, None], (NBLK, C))], axis=1)
    o_ref[...] = acc * r + skip_ref[...]


def _sc_gather_body(q_hbm, kv_hbm, src_hbm, dst_hbm, qd_hbm, kvs_hbm,
                    idxs_v, idxd_v, qrows_v, kvrows_v, sem1, sem2):
    wid = lax.axis_index("s") * NC + lax.axis_index("c")
    base = wid * EPW

    def win(g, carry):
        b = pl.multiple_of(base + g * GW, 8)
        pltpu.sync_copy(src_hbm.at[pl.ds(b, GW)], idxs_v)
        pltpu.sync_copy(dst_hbm.at[pl.ds(b, GW)], idxd_v)
        c1 = pltpu.async_copy(q_hbm.at[idxd_v], qrows_v, sem1)
        c2 = pltpu.async_copy(kv_hbm.at[idxs_v], kvrows_v, sem2)
        c1.wait()
        c2.wait()
        pltpu.sync_copy(qrows_v, qd_hbm.at[pl.ds(b, GW)])
        pltpu.sync_copy(kvrows_v, kvs_hbm.at[pl.ds(b, GW)])
        return carry

    lax.fori_loop(0, NWIN, win, 0)


def _sc_gather(q, kv, src, dst):
    return pl.kernel(
        _sc_gather_body,
        out_type=[
            jax.ShapeDtypeStruct((E, HC), jnp.float32),
            jax.ShapeDtypeStruct((E, 2 * HC), jnp.float32),
        ],
        mesh=plsc.VectorSubcoreMesh(core_axis_name="c", subcore_axis_name="s"),
        scratch_types=[
            pltpu.VMEM((GW,), jnp.int32),
            pltpu.VMEM((GW,), jnp.int32),
            pltpu.VMEM((GW, HC), jnp.float32),
            pltpu.VMEM((GW, 2 * HC), jnp.float32),
            pltpu.SemaphoreType.DMA,
            pltpu.SemaphoreType.DMA,
        ],
    )(q, kv, src, dst)


def _sc_scatter_body(wv_hbm, ex0_hbm, ex1_hbm, dst_hbm, z2_hbm, z1_hbm,
                     part_hbm, den_hbm,
                     idxd_v, wrows_v, ex0_v, ex1_v,
                     out_sh, den0_sh, den1_sh, sem1):
    cid = lax.axis_index("c")
    sid = lax.axis_index("s")
    wid = sid * NC + cid
    base = wid * EPW

    pltpu.sync_copy(z2_hbm.at[pl.ds(sid * NPT, NPT)],
                    out_sh.at[pl.ds(sid * NPT, NPT)])
    pltpu.sync_copy(z1_hbm.at[pl.ds(sid * NPT, NPT)],
                    den0_sh.at[pl.ds(sid * NPT, NPT)])
    pltpu.sync_copy(z1_hbm.at[pl.ds(sid * NPT, NPT)],
                    den1_sh.at[pl.ds(sid * NPT, NPT)])
    plsc.subcore_barrier()

    def win(g, carry):
        b = pl.multiple_of(base + g * GW, 8)
        pltpu.sync_copy(dst_hbm.at[pl.ds(b, GW)], idxd_v.at[0])
        c1 = pltpu.async_copy(wv_hbm.at[pl.ds(b, GW)], wrows_v, sem1)
        pltpu.sync_copy(ex0_hbm.at[pl.ds(b, GW)], ex0_v)
        pltpu.sync_copy(ex1_hbm.at[pl.ds(b, GW)], ex1_v)
        c1.wait()

        pltpu.sync_copy(wrows_v, out_sh.at[idxd_v.at[0]], add=True)
        pltpu.sync_copy(ex0_v, den0_sh.at[idxd_v.at[0]], add=True)
        pltpu.sync_copy(ex1_v, den1_sh.at[idxd_v.at[0]], add=True)
        return carry

    lax.fori_loop(0, NWIN, win, 0)
    plsc.subcore_barrier()

    pltpu.sync_copy(out_sh.at[pl.ds(sid * NPT, NPT)],
                    part_hbm.at[cid, pl.ds(sid * NPT, NPT)])
    pltpu.sync_copy(den0_sh.at[pl.ds(sid * NPT, NPT)],
                    den_hbm.at[cid, 0, pl.ds(sid * NPT, NPT)])
    pltpu.sync_copy(den1_sh.at[pl.ds(sid * NPT, NPT)],
                    den_hbm.at[cid, 1, pl.ds(sid * NPT, NPT)])


def _sc_scatter(wv, ex0, ex1, dst):
    z2 = jnp.zeros((NPAD, HC), jnp.float32)
    z1 = jnp.zeros((NPAD,), jnp.float32)
    return pl.kernel(
        _sc_scatter_body,
        out_type=[
            jax.ShapeDtypeStruct((NC, NPAD, HC), jnp.float32),
            jax.ShapeDtypeStruct((NC, 2, NPAD), jnp.float32),
        ],
        mesh=plsc.VectorSubcoreMesh(core_axis_name="c", subcore_axis_name="s"),
        scratch_types=[
            pltpu.VMEM((1, GW), jnp.int32),
            pltpu.VMEM((GW, HC), jnp.float32),
            pltpu.VMEM((GW,), jnp.float32),
            pltpu.VMEM((GW,), jnp.float32),
            pltpu.VMEM_SHARED((NPAD, HC), jnp.float32),
            pltpu.VMEM_SHARED((NPAD,), jnp.float32),
            pltpu.VMEM_SHARED((NPAD,), jnp.float32),
            pltpu.SemaphoreType.DMA,
        ],
    )(wv, ex0, ex1, dst, z2, z1)


def kernel(x, edge_index, edge_time, msg, time_w, time_b, Wq, bq, Wk, bk, Wv, bv, We, Wskip, bskip):
    Wall = jnp.concatenate([Wq.T, Wk.T, Wv.T, Wskip.T], axis=1)
    ball = jnp.concatenate([bq, bk, bv, bskip])[None, :]

    qkvg = pl.pallas_call(
        _qkvg_body,
        grid=(N // 1000,),
        in_specs=[
            pl.BlockSpec((1000, D), lambda i: (i, 0)),
            pl.BlockSpec((D, 4 * HC), lambda i: (0, 0)),
            pl.BlockSpec((1, 4 * HC), lambda i: (0, 0)),
        ],
        out_specs=pl.BlockSpec((1000, 4 * HC), lambda i: (i, 0)),
        out_shape=jax.ShapeDtypeStruct((N, 4 * HC), jnp.float32),
    )(x, Wall, ball)
    q = qkvg[:, :HC]
    kv = qkvg[:, HC:3 * HC]
    skip = qkvg[:, 3 * HC:]

    src = edge_index[0]
    dst = edge_index[1]
    qd, kvs = _sc_gather(q, kv, src, dst)

    wt = jnp.pad(time_w, (0, HC - TD))[None, :]
    bt = jnp.pad(time_b, (0, HC - TD))[None, :]
    WeTp = jnp.pad(We.T, ((0, HC - TD), (0, 0)))
    t2 = edge_time[:, None]

    ex0, ex1, wv = pl.pallas_call(
        _alpha_body,
        grid=(E // EBLK,),
        in_specs=[
            pl.BlockSpec((EBLK, HC), lambda i: (i, 0)),
            pl.BlockSpec((EBLK, 2 * HC), lambda i: (i, 0)),
            pl.BlockSpec((EBLK, 1), lambda i: (i, 0)),
            pl.BlockSpec((1, HC), lambda i: (0, 0)),
            pl.BlockSpec((1, HC), lambda i: (0, 0)),
            pl.BlockSpec((HC, HC), lambda i: (0, 0)),
        ],
        out_specs=[
            pl.BlockSpec((1, EBLK), lambda i: (0, i)),
            pl.BlockSpec((1, EBLK), lambda i: (0, i)),
            pl.BlockSpec((EBLK, HC), lambda i: (i, 0)),
        ],
        out_shape=[
            jax.ShapeDtypeStruct((1, E), jnp.float32),
            jax.ShapeDtypeStruct((1, E), jnp.float32),
            jax.ShapeDtypeStruct((E, HC), jnp.float32),
        ],
    )(qd, kvs, t2, wt, bt, WeTp)

    part, den = _sc_scatter(wv, ex0.reshape(E), ex1.reshape(E), dst)

    skip_pad = jnp.pad(skip, ((0, NPAD - N), (0, 0)))
    out_pad = pl.pallas_call(
        _final_body,
        grid=(NPAD // NBLK,),
        in_specs=[
            pl.BlockSpec((NC, NBLK, HC), lambda i: (0, i, 0)),
            pl.BlockSpec((NC, 2, NBLK), lambda i: (0, 0, i)),
            pl.BlockSpec((NBLK, HC), lambda i: (i, 0)),
        ],
        out_specs=pl.BlockSpec((NBLK, HC), lambda i: (i, 0)),
        out_shape=jax.ShapeDtypeStruct((NPAD, HC), jnp.float32),
    )(part, den, skip_pad)
    return out_pad[:N]


# speedup vs baseline: 1.1485x; 1.1485x over previous
import functools

import jax
import jax.numpy as jnp
from jax import lax
from jax.experimental import pallas as pl
from jax.experimental.pallas import tpu as pltpu
from jax.experimental.pallas import tpu_sc as plsc

N = 10000
E = 320000
D = 128
H = 2
C = 64
HC = H * C
TD = 100

NPAD = 10240
NBLK = 1024
EBLK = 512

_SC_INFO = plsc.get_sparse_core_info()
NC = _SC_INFO.num_cores
NS = _SC_INFO.num_subcores
NW = NC * NS
EPW = E // NW
GW = 80
NWIN = EPW // GW
NPT = NPAD // NS


def _qkvg_body(x_ref, w_ref, b_ref, q_ref, kvp_ref, g_ref):
    o = (jnp.dot(x_ref[...], w_ref[...], preferred_element_type=jnp.float32)
         + b_ref[...])
    u16 = jnp.uint16
    u32 = jnp.uint32
    kb = lax.bitcast_convert_type(o[:, HC:2 * HC].astype(jnp.bfloat16), u16)
    vb = lax.bitcast_convert_type(o[:, 2 * HC:3 * HC].astype(jnp.bfloat16), u16)
    kvp = (kb.astype(u32) << 16) | vb.astype(u32)
    q_ref[...] = o[:, :HC]
    kvp_ref[...] = lax.bitcast_convert_type(kvp, jnp.float32)
    g_ref[...] = o[:, 3 * HC:]


def _time_e_body(t_ref, wt_ref, bt_ref, we_ref, e_ref):
    enc = jnp.cos(t_ref[...] * wt_ref[...] + bt_ref[...])
    e_ref[...] = jnp.dot(enc, we_ref[...], preferred_element_type=jnp.float32)


def _unpack(u, hi):
    u16 = jnp.uint16
    h = (u >> 16).astype(u16) if hi else (u & 0xFFFF).astype(u16)
    return lax.bitcast_convert_type(h, jnp.bfloat16).astype(jnp.float32)


def _alpha_body(qd_ref, kvs_ref, e_ref, ex0_ref, ex1_ref, wv_ref):
    eb = e_ref[...]
    kv_u = lax.bitcast_convert_type(kvs_ref[...], jnp.uint32)
    ks = _unpack(kv_u, True)
    vs = _unpack(kv_u, False)
    a = qd_ref[...] * (ks + eb)
    h0 = jnp.sum(a[:, :C], axis=1) * (1.0 / 8.0)
    h1 = jnp.sum(a[:, C:], axis=1) * (1.0 / 8.0)
    ex0 = jnp.exp(h0)
    ex1 = jnp.exp(h1)
    ex0_ref[...] = ex0[None, :]
    ex1_ref[...] = ex1[None, :]
    w = jnp.concatenate(
        [jnp.broadcast_to(ex0[:, None], (EBLK, C)),
         jnp.broadcast_to(ex1[:, None], (EBLK, C))], axis=1)
    wv_ref[...] = (vs + eb) * w


def _final_body(p_ref, den_ref, skip_ref, o_ref):
    acc = p_ref[0] + p_ref[1]
    d0 = den_ref[0, 0] + den_ref[1, 0]
    d1 = den_ref[0, 1] + den_ref[1, 1]
    r0 = 1.0 / (d0 + 1e-16)
    r1 = 1.0 / (d1 + 1e-16)
    r = jnp.concatenate(
        [jnp.broadcast_to(r0[:, None], (NBLK, C)),
         jnp.broadcast_to(r1[:, None], (NBLK, C))], axis=1)
    o_ref[...] = acc * r + skip_ref[...]


def _sc_gather_body(q_hbm, kv_hbm, src_hbm, dst_hbm, qd_hbm, kvs_hbm,
                    idxs_v, idxd_v, qrows_v, kvrows_v, sem1, sem2):
    wid = lax.axis_index("s") * NC + lax.axis_index("c")
    base = wid * EPW

    def win(g, carry):
        b = pl.multiple_of(base + g * GW, 8)
        pltpu.sync_copy(src_hbm.at[pl.ds(b, GW)], idxs_v)
        pltpu.sync_copy(dst_hbm.at[pl.ds(b, GW)], idxd_v)
        c1 = pltpu.async_copy(q_hbm.at[idxd_v], qrows_v, sem1)
        c2 = pltpu.async_copy(kv_hbm.at[idxs_v], kvrows_v, sem2)
        c1.wait()
        c2.wait()
        pltpu.sync_copy(qrows_v, qd_hbm.at[pl.ds(b, GW)])
        pltpu.sync_copy(kvrows_v, kvs_hbm.at[pl.ds(b, GW)])
        return carry

    lax.fori_loop(0, NWIN, win, 0)


def _sc_gather(q, kvp, src, dst):
    return pl.kernel(
        _sc_gather_body,
        out_type=[
            jax.ShapeDtypeStruct((E, HC), jnp.float32),
            jax.ShapeDtypeStruct((E, HC), jnp.float32),
        ],
        mesh=plsc.VectorSubcoreMesh(core_axis_name="c", subcore_axis_name="s"),
        scratch_types=[
            pltpu.VMEM((GW,), jnp.int32),
            pltpu.VMEM((GW,), jnp.int32),
            pltpu.VMEM((GW, HC), jnp.float32),
            pltpu.VMEM((GW, HC), jnp.float32),
            pltpu.SemaphoreType.DMA,
            pltpu.SemaphoreType.DMA,
        ],
    )(q, kvp, src, dst)


def _sc_scatter_body(wv_hbm, ex0_hbm, ex1_hbm, dst_hbm, z2_hbm, z1_hbm,
                     part_hbm, den_hbm,
                     idxd_v, wrows_v, ex0_v, ex1_v,
                     out_sh, den0_sh, den1_sh, sem1):
    cid = lax.axis_index("c")
    sid = lax.axis_index("s")
    wid = sid * NC + cid
    base = wid * EPW

    pltpu.sync_copy(z2_hbm.at[pl.ds(sid * NPT, NPT)],
                    out_sh.at[pl.ds(sid * NPT, NPT)])
    pltpu.sync_copy(z1_hbm.at[pl.ds(sid * NPT, NPT)],
                    den0_sh.at[pl.ds(sid * NPT, NPT)])
    pltpu.sync_copy(z1_hbm.at[pl.ds(sid * NPT, NPT)],
                    den1_sh.at[pl.ds(sid * NPT, NPT)])
    plsc.subcore_barrier()

    def win(g, carry):
        b = pl.multiple_of(base + g * GW, 8)
        pltpu.sync_copy(dst_hbm.at[pl.ds(b, GW)], idxd_v.at[0])
        c1 = pltpu.async_copy(wv_hbm.at[pl.ds(b, GW)], wrows_v, sem1)
        pltpu.sync_copy(ex0_hbm.at[pl.ds(b, GW)], ex0_v)
        pltpu.sync_copy(ex1_hbm.at[pl.ds(b, GW)], ex1_v)
        c1.wait()

        pltpu.sync_copy(wrows_v, out_sh.at[idxd_v.at[0]], add=True)
        pltpu.sync_copy(ex0_v, den0_sh.at[idxd_v.at[0]], add=True)
        pltpu.sync_copy(ex1_v, den1_sh.at[idxd_v.at[0]], add=True)
        return carry

    lax.fori_loop(0, NWIN, win, 0)
    plsc.subcore_barrier()

    pltpu.sync_copy(out_sh.at[pl.ds(sid * NPT, NPT)],
                    part_hbm.at[cid, pl.ds(sid * NPT, NPT)])
    pltpu.sync_copy(den0_sh.at[pl.ds(sid * NPT, NPT)],
                    den_hbm.at[cid, 0, pl.ds(sid * NPT, NPT)])
    pltpu.sync_copy(den1_sh.at[pl.ds(sid * NPT, NPT)],
                    den_hbm.at[cid, 1, pl.ds(sid * NPT, NPT)])


def _sc_scatter(wv, ex0, ex1, dst):
    z2 = jnp.zeros((NPAD, HC), jnp.float32)
    z1 = jnp.zeros((NPAD,), jnp.float32)
    return pl.kernel(
        _sc_scatter_body,
        out_type=[
            jax.ShapeDtypeStruct((NC, NPAD, HC), jnp.float32),
            jax.ShapeDtypeStruct((NC, 2, NPAD), jnp.float32),
        ],
        mesh=plsc.VectorSubcoreMesh(core_axis_name="c", subcore_axis_name="s"),
        scratch_types=[
            pltpu.VMEM((1, GW), jnp.int32),
            pltpu.VMEM((GW, HC), jnp.float32),
            pltpu.VMEM((GW,), jnp.float32),
            pltpu.VMEM((GW,), jnp.float32),
            pltpu.VMEM_SHARED((NPAD, HC), jnp.float32),
            pltpu.VMEM_SHARED((NPAD,), jnp.float32),
            pltpu.VMEM_SHARED((NPAD,), jnp.float32),
            pltpu.SemaphoreType.DMA,
        ],
    )(wv, ex0, ex1, dst, z2, z1)


def kernel(x, edge_index, edge_time, msg, time_w, time_b, Wq, bq, Wk, bk, Wv, bv, We, Wskip, bskip):
    Wall = jnp.concatenate([Wq.T, Wk.T, Wv.T, Wskip.T], axis=1)
    ball = jnp.concatenate([bq, bk, bv, bskip])[None, :]

    q, kvp, skip = pl.pallas_call(
        _qkvg_body,
        grid=(N // 1000,),
        in_specs=[
            pl.BlockSpec((1000, D), lambda i: (i, 0)),
            pl.BlockSpec((D, 4 * HC), lambda i: (0, 0)),
            pl.BlockSpec((1, 4 * HC), lambda i: (0, 0)),
        ],
        out_specs=[
            pl.BlockSpec((1000, HC), lambda i: (i, 0)),
            pl.BlockSpec((1000, HC), lambda i: (i, 0)),
            pl.BlockSpec((1000, HC), lambda i: (i, 0)),
        ],
        out_shape=[
            jax.ShapeDtypeStruct((N, HC), jnp.float32),
            jax.ShapeDtypeStruct((N, HC), jnp.float32),
            jax.ShapeDtypeStruct((N, HC), jnp.float32),
        ],
    )(x, Wall, ball)

    wt = jnp.pad(time_w, (0, HC - TD))[None, :]
    bt = jnp.pad(time_b, (0, HC - TD))[None, :]
    WeTp = jnp.pad(We.T, ((0, HC - TD), (0, 0)))
    t2 = edge_time[:, None]

    e = pl.pallas_call(
        _time_e_body,
        grid=(E // EBLK,),
        in_specs=[
            pl.BlockSpec((EBLK, 1), lambda i: (i, 0)),
            pl.BlockSpec((1, HC), lambda i: (0, 0)),
            pl.BlockSpec((1, HC), lambda i: (0, 0)),
            pl.BlockSpec((HC, HC), lambda i: (0, 0)),
        ],
        out_specs=pl.BlockSpec((EBLK, HC), lambda i: (i, 0)),
        out_shape=jax.ShapeDtypeStruct((E, HC), jnp.float32),
    )(t2, wt, bt, WeTp)

    src = edge_index[0]
    dst = edge_index[1]
    qd, kvs = _sc_gather(q, kvp, src, dst)

    ex0, ex1, wv = pl.pallas_call(
        _alpha_body,
        grid=(E // EBLK,),
        in_specs=[
            pl.BlockSpec((EBLK, HC), lambda i: (i, 0)),
            pl.BlockSpec((EBLK, HC), lambda i: (i, 0)),
            pl.BlockSpec((EBLK, HC), lambda i: (i, 0)),
        ],
        out_specs=[
            pl.BlockSpec((1, EBLK), lambda i: (0, i)),
            pl.BlockSpec((1, EBLK), lambda i: (0, i)),
            pl.BlockSpec((EBLK, HC), lambda i: (i, 0)),
        ],
        out_shape=[
            jax.ShapeDtypeStruct((1, E), jnp.float32),
            jax.ShapeDtypeStruct((1, E), jnp.float32),
            jax.ShapeDtypeStruct((E, HC), jnp.float32),
        ],
    )(qd, kvs, e)

    part, den = _sc_scatter(wv, ex0.reshape(E), ex1.reshape(E), dst)

    skip_pad = jnp.pad(skip, ((0, NPAD - N), (0, 0)))
    out_pad = pl.pallas_call(
        _final_body,
        grid=(NPAD // NBLK,),
        in_specs=[
            pl.BlockSpec((NC, NBLK, HC), lambda i: (0, i, 0)),
            pl.BlockSpec((NC, 2, NBLK), lambda i: (0, 0, i)),
            pl.BlockSpec((NBLK, HC), lambda i: (i, 0)),
        ],
        out_specs=pl.BlockSpec((NBLK, HC), lambda i: (i, 0)),
        out_shape=jax.ShapeDtypeStruct((NPAD, HC), jnp.float32),
    )(part, den, skip_pad)
    return out_pad[:N]


# speedup vs baseline: 1.2315x; 1.0723x over previous
import functools

import jax
import jax.numpy as jnp
from jax import lax
from jax.experimental import pallas as pl
from jax.experimental.pallas import tpu as pltpu
from jax.experimental.pallas import tpu_sc as plsc

N = 10000
E = 320000
D = 128
H = 2
C = 64
HC = H * C
TD = 100

NPAD = 10240
NBLK = 1024
EBLK = 512

_SC_INFO = plsc.get_sparse_core_info()
NC = _SC_INFO.num_cores
NS = _SC_INFO.num_subcores
NW = NC * NS
EPW = E // NW
GW = 80
NWIN = EPW // GW
NPT = NPAD // NS


def _qkvg_body(x_ref, w_ref, b_ref, q_ref, kvp_ref, g_ref):
    o = (jnp.dot(x_ref[...], w_ref[...], preferred_element_type=jnp.float32)
         + b_ref[...])
    u16 = jnp.uint16
    u32 = jnp.uint32
    kb = lax.bitcast_convert_type(o[:, HC:2 * HC].astype(jnp.bfloat16), u16)
    vb = lax.bitcast_convert_type(o[:, 2 * HC:3 * HC].astype(jnp.bfloat16), u16)
    kvp = (kb.astype(u32) << 16) | vb.astype(u32)
    q_ref[...] = o[:, :HC]
    kvp_ref[...] = lax.bitcast_convert_type(kvp, jnp.float32)
    g_ref[...] = o[:, 3 * HC:]


def _time_e_body(t_ref, wt_ref, bt_ref, we_ref, e_ref):
    enc = jnp.cos(t_ref[...] * wt_ref[...] + bt_ref[...])
    e_ref[...] = jnp.dot(enc, we_ref[...], preferred_element_type=jnp.float32)


def _unpack(u, hi):
    u16 = jnp.uint16
    h = (u >> 16).astype(u16) if hi else (u & 0xFFFF).astype(u16)
    return lax.bitcast_convert_type(h, jnp.bfloat16).astype(jnp.float32)


def _alpha_body(qd_ref, kvs_ref, e_ref, ex0_ref, ex1_ref, wv_ref):
    eb = e_ref[...]
    kv_u = lax.bitcast_convert_type(kvs_ref[...], jnp.uint32)
    ks = _unpack(kv_u, True)
    vs = _unpack(kv_u, False)
    a = qd_ref[...] * (ks + eb)
    h0 = jnp.sum(a[:, :C], axis=1) * (1.0 / 8.0)
    h1 = jnp.sum(a[:, C:], axis=1) * (1.0 / 8.0)
    ex0 = jnp.exp(h0)
    ex1 = jnp.exp(h1)
    ex0_ref[...] = ex0[None, :]
    ex1_ref[...] = ex1[None, :]
    w = jnp.concatenate(
        [jnp.broadcast_to(ex0[:, None], (EBLK, C)),
         jnp.broadcast_to(ex1[:, None], (EBLK, C))], axis=1)
    wv_ref[...] = (vs + eb) * w


def _final_body(p_ref, den_ref, skip_ref, o_ref):
    acc = p_ref[0] + p_ref[1]
    d0 = den_ref[0, 0] + den_ref[1, 0]
    d1 = den_ref[0, 1] + den_ref[1, 1]
    r0 = 1.0 / (d0 + 1e-16)
    r1 = 1.0 / (d1 + 1e-16)
    r = jnp.concatenate(
        [jnp.broadcast_to(r0[:, None], (NBLK, C)),
         jnp.broadcast_to(r1[:, None], (NBLK, C))], axis=1)
    o_ref[...] = acc * r + skip_ref[...]


def _sc_gather_body(q_hbm, kv_hbm, src_hbm, dst_hbm, qd_hbm, kvs_hbm,
                    idxs_v, idxd_v, qrows_v, kvrows_v, sem1, sem2):
    wid = lax.axis_index("s") * NC + lax.axis_index("c")
    base = wid * EPW

    def win(g, carry):
        b = pl.multiple_of(base + g * GW, 8)
        pltpu.sync_copy(src_hbm.at[pl.ds(b, GW)], idxs_v)
        pltpu.sync_copy(dst_hbm.at[pl.ds(b, GW)], idxd_v)
        c1 = pltpu.async_copy(q_hbm.at[idxd_v], qrows_v, sem1)
        c2 = pltpu.async_copy(kv_hbm.at[idxs_v], kvrows_v, sem2)
        c1.wait()
        c2.wait()
        pltpu.sync_copy(qrows_v, qd_hbm.at[pl.ds(b, GW)])
        pltpu.sync_copy(kvrows_v, kvs_hbm.at[pl.ds(b, GW)])
        return carry

    lax.fori_loop(0, NWIN, win, 0)


def _sc_gather(q, kvp, src, dst):
    return pl.kernel(
        _sc_gather_body,
        out_type=[
            jax.ShapeDtypeStruct((E, HC), jnp.float32),
            jax.ShapeDtypeStruct((E, HC), jnp.float32),
        ],
        mesh=plsc.VectorSubcoreMesh(core_axis_name="c", subcore_axis_name="s"),
        scratch_types=[
            pltpu.VMEM((GW,), jnp.int32),
            pltpu.VMEM((GW,), jnp.int32),
            pltpu.VMEM((GW, HC), jnp.float32),
            pltpu.VMEM((GW, HC), jnp.float32),
            pltpu.SemaphoreType.DMA,
            pltpu.SemaphoreType.DMA,
        ],
    )(q, kvp, src, dst)


def _sc_scatter_body(wv_hbm, ex0_hbm, ex1_hbm, dst_hbm, z2_hbm, z1_hbm,
                     part_hbm, den_hbm,
                     idxd_v, wrows_v, ex0_v, ex1_v,
                     idxd2_v, wrows2_v, ex02_v, ex12_v,
                     out_sh, den0_sh, den1_sh,
                     smi0, smw0, sm00, sm10, smi1, smw1, sm01, sm11):
    cid = lax.axis_index("c")
    sid = lax.axis_index("s")
    wid = sid * NC + cid
    base = wid * EPW

    pltpu.sync_copy(z2_hbm.at[pl.ds(sid * NPT, NPT)],
                    out_sh.at[pl.ds(sid * NPT, NPT)])
    pltpu.sync_copy(z1_hbm.at[pl.ds(sid * NPT, NPT)],
                    den0_sh.at[pl.ds(sid * NPT, NPT)])
    pltpu.sync_copy(z1_hbm.at[pl.ds(sid * NPT, NPT)],
                    den1_sh.at[pl.ds(sid * NPT, NPT)])
    plsc.subcore_barrier()

    def load(g, idxd, wr, e0, e1, si, sw, s0, s1):
        b = pl.multiple_of(base + g * GW, 8)
        pltpu.async_copy(dst_hbm.at[pl.ds(b, GW)], idxd.at[0], si)
        pltpu.async_copy(wv_hbm.at[pl.ds(b, GW)], wr, sw)
        pltpu.async_copy(ex0_hbm.at[pl.ds(b, GW)], e0, s0)
        pltpu.async_copy(ex1_hbm.at[pl.ds(b, GW)], e1, s1)

    def wait_l(g, idxd, wr, e0, e1, si, sw, s0, s1):
        b = pl.multiple_of(base + g * GW, 8)
        pltpu.make_async_copy(dst_hbm.at[pl.ds(b, GW)], idxd.at[0], si).wait()
        pltpu.make_async_copy(wv_hbm.at[pl.ds(b, GW)], wr, sw).wait()
        pltpu.make_async_copy(ex0_hbm.at[pl.ds(b, GW)], e0, s0).wait()
        pltpu.make_async_copy(ex1_hbm.at[pl.ds(b, GW)], e1, s1).wait()

    def scat(idxd, wr, e0, e1):
        pltpu.sync_copy(wr, out_sh.at[idxd.at[0]], add=True)
        pltpu.sync_copy(e0, den0_sh.at[idxd.at[0]], add=True)
        pltpu.sync_copy(e1, den1_sh.at[idxd.at[0]], add=True)

    set0 = (idxd_v, wrows_v, ex0_v, ex1_v, smi0, smw0, sm00, sm10)
    set1 = (idxd2_v, wrows2_v, ex02_v, ex12_v, smi1, smw1, sm01, sm11)

    load(0, *set0)

    def pair(h, carry):
        g0 = 2 * h
        g1 = g0 + 1
        wait_l(g0, *set0)
        load(g1, *set1)
        scat(idxd_v, wrows_v, ex0_v, ex1_v)
        wait_l(g1, *set1)

        @pl.when(g0 + 2 < NWIN)
        def _():
            load(g0 + 2, *set0)

        scat(idxd2_v, wrows2_v, ex02_v, ex12_v)
        return carry

    lax.fori_loop(0, NWIN // 2, pair, 0)
    wait_l(NWIN - 1, *set0)
    scat(idxd_v, wrows_v, ex0_v, ex1_v)
    plsc.subcore_barrier()

    pltpu.sync_copy(out_sh.at[pl.ds(sid * NPT, NPT)],
                    part_hbm.at[cid, pl.ds(sid * NPT, NPT)])
    pltpu.sync_copy(den0_sh.at[pl.ds(sid * NPT, NPT)],
                    den_hbm.at[cid, 0, pl.ds(sid * NPT, NPT)])
    pltpu.sync_copy(den1_sh.at[pl.ds(sid * NPT, NPT)],
                    den_hbm.at[cid, 1, pl.ds(sid * NPT, NPT)])


def _sc_scatter(wv, ex0, ex1, dst):
    z2 = jnp.zeros((NPAD, HC), jnp.float32)
    z1 = jnp.zeros((NPAD,), jnp.float32)
    return pl.kernel(
        _sc_scatter_body,
        out_type=[
            jax.ShapeDtypeStruct((NC, NPAD, HC), jnp.float32),
            jax.ShapeDtypeStruct((NC, 2, NPAD), jnp.float32),
        ],
        mesh=plsc.VectorSubcoreMesh(core_axis_name="c", subcore_axis_name="s"),
        scratch_types=[
            pltpu.VMEM((1, GW), jnp.int32),
            pltpu.VMEM((GW, HC), jnp.float32),
            pltpu.VMEM((GW,), jnp.float32),
            pltpu.VMEM((GW,), jnp.float32),
            pltpu.VMEM((1, GW), jnp.int32),
            pltpu.VMEM((GW, HC), jnp.float32),
            pltpu.VMEM((GW,), jnp.float32),
            pltpu.VMEM((GW,), jnp.float32),
            pltpu.VMEM_SHARED((NPAD, HC), jnp.float32),
            pltpu.VMEM_SHARED((NPAD,), jnp.float32),
            pltpu.VMEM_SHARED((NPAD,), jnp.float32),
        ] + [pltpu.SemaphoreType.DMA] * 8,
    )(wv, ex0, ex1, dst, z2, z1)


def kernel(x, edge_index, edge_time, msg, time_w, time_b, Wq, bq, Wk, bk, Wv, bv, We, Wskip, bskip):
    Wall = jnp.concatenate([Wq.T, Wk.T, Wv.T, Wskip.T], axis=1)
    ball = jnp.concatenate([bq, bk, bv, bskip])[None, :]

    q, kvp, skip = pl.pallas_call(
        _qkvg_body,
        grid=(N // 1000,),
        in_specs=[
            pl.BlockSpec((1000, D), lambda i: (i, 0)),
            pl.BlockSpec((D, 4 * HC), lambda i: (0, 0)),
            pl.BlockSpec((1, 4 * HC), lambda i: (0, 0)),
        ],
        out_specs=[
            pl.BlockSpec((1000, HC), lambda i: (i, 0)),
            pl.BlockSpec((1000, HC), lambda i: (i, 0)),
            pl.BlockSpec((1000, HC), lambda i: (i, 0)),
        ],
        out_shape=[
            jax.ShapeDtypeStruct((N, HC), jnp.float32),
            jax.ShapeDtypeStruct((N, HC), jnp.float32),
            jax.ShapeDtypeStruct((N, HC), jnp.float32),
        ],
    )(x, Wall, ball)

    wt = jnp.pad(time_w, (0, HC - TD))[None, :]
    bt = jnp.pad(time_b, (0, HC - TD))[None, :]
    WeTp = jnp.pad(We.T, ((0, HC - TD), (0, 0)))
    t2 = edge_time[:, None]

    e = pl.pallas_call(
        _time_e_body,
        grid=(E // EBLK,),
        in_specs=[
            pl.BlockSpec((EBLK, 1), lambda i: (i, 0)),
            pl.BlockSpec((1, HC), lambda i: (0, 0)),
            pl.BlockSpec((1, HC), lambda i: (0, 0)),
            pl.BlockSpec((HC, HC), lambda i: (0, 0)),
        ],
        out_specs=pl.BlockSpec((EBLK, HC), lambda i: (i, 0)),
        out_shape=jax.ShapeDtypeStruct((E, HC), jnp.float32),
    )(t2, wt, bt, WeTp)

    src = edge_index[0]
    dst = edge_index[1]
    qd, kvs = _sc_gather(q, kvp, src, dst)

    ex0, ex1, wv = pl.pallas_call(
        _alpha_body,
        grid=(E // EBLK,),
        in_specs=[
            pl.BlockSpec((EBLK, HC), lambda i: (i, 0)),
            pl.BlockSpec((EBLK, HC), lambda i: (i, 0)),
            pl.BlockSpec((EBLK, HC), lambda i: (i, 0)),
        ],
        out_specs=[
            pl.BlockSpec((1, EBLK), lambda i: (0, i)),
            pl.BlockSpec((1, EBLK), lambda i: (0, i)),
            pl.BlockSpec((EBLK, HC), lambda i: (i, 0)),
        ],
        out_shape=[
            jax.ShapeDtypeStruct((1, E), jnp.float32),
            jax.ShapeDtypeStruct((1, E), jnp.float32),
            jax.ShapeDtypeStruct((E, HC), jnp.float32),
        ],
    )(qd, kvs, e)

    part, den = _sc_scatter(wv, ex0.reshape(E), ex1.reshape(E), dst)

    skip_pad = jnp.pad(skip, ((0, NPAD - N), (0, 0)))
    out_pad = pl.pallas_call(
        _final_body,
        grid=(NPAD // NBLK,),
        in_specs=[
            pl.BlockSpec((NC, NBLK, HC), lambda i: (0, i, 0)),
            pl.BlockSpec((NC, 2, NBLK), lambda i: (0, 0, i)),
            pl.BlockSpec((NBLK, HC), lambda i: (i, 0)),
        ],
        out_specs=pl.BlockSpec((NBLK, HC), lambda i: (i, 0)),
        out_shape=jax.ShapeDtypeStruct((NPAD, HC), jnp.float32),
    )(part, den, skip_pad)
    return out_pad[:N]


# speedup vs baseline: 1.2337x; 1.0018x over previous
import functools

import jax
import jax.numpy as jnp
from jax import lax
from jax.experimental import pallas as pl
from jax.experimental.pallas import tpu as pltpu
from jax.experimental.pallas import tpu_sc as plsc

N = 10000
E = 320000
D = 128
H = 2
C = 64
HC = H * C
TD = 100

NPAD = 10240
NBLK = 1024
EBLK = 512

_SC_INFO = plsc.get_sparse_core_info()
NC = _SC_INFO.num_cores
NS = _SC_INFO.num_subcores
NW = NC * NS
EPW = E // NW
GW = 80
NWIN = EPW // GW
NPT = NPAD // NS


def _qkvg_body(x_ref, w_ref, b_ref, q_ref, kvp_ref, g_ref):
    o = (jnp.dot(x_ref[...], w_ref[...], preferred_element_type=jnp.float32)
         + b_ref[...])
    u32 = jnp.uint32
    kb = lax.bitcast_convert_type(
        o[:, HC:2 * HC].astype(jnp.bfloat16).astype(jnp.float32), u32)
    vb = lax.bitcast_convert_type(
        o[:, 2 * HC:3 * HC].astype(jnp.bfloat16).astype(jnp.float32), u32)
    kvp = (kb & jnp.uint32(0xFFFF0000)) | (vb >> 16)
    q_ref[...] = o[:, :HC]
    kvp_ref[...] = lax.bitcast_convert_type(kvp, jnp.float32)
    g_ref[...] = o[:, 3 * HC:]


def _time_e_body(t_ref, wt_ref, bt_ref, we_ref, e_ref):
    enc = jnp.cos(t_ref[...] * wt_ref[...] + bt_ref[...])
    e_ref[...] = jnp.dot(enc.astype(jnp.bfloat16), we_ref[...],
                         preferred_element_type=jnp.float32)


def _unpack(u, hi):
    h = (u & jnp.uint32(0xFFFF0000)) if hi else (u << 16)
    return lax.bitcast_convert_type(h, jnp.float32)


def _alpha_body(qd_ref, kvs_ref, e_ref, ex0_ref, ex1_ref, wv_ref):
    eb = e_ref[...]
    kv_u = lax.bitcast_convert_type(kvs_ref[...], jnp.uint32)
    ks = _unpack(kv_u, True)
    vs = _unpack(kv_u, False)
    a = qd_ref[...] * (ks + eb)
    h0 = jnp.sum(a[:, :C], axis=1) * (1.0 / 8.0)
    h1 = jnp.sum(a[:, C:], axis=1) * (1.0 / 8.0)
    ex0 = jnp.exp(h0)
    ex1 = jnp.exp(h1)
    ex0_ref[...] = ex0[None, :]
    ex1_ref[...] = ex1[None, :]
    w = jnp.concatenate(
        [jnp.broadcast_to(ex0[:, None], (EBLK, C)),
         jnp.broadcast_to(ex1[:, None], (EBLK, C))], axis=1)
    wv_ref[...] = (vs + eb) * w


def _final_body(p_ref, den_ref, skip_ref, o_ref):
    acc = p_ref[0] + p_ref[1]
    d0 = den_ref[0, 0] + den_ref[1, 0]
    d1 = den_ref[0, 1] + den_ref[1, 1]
    r0 = 1.0 / (d0 + 1e-16)
    r1 = 1.0 / (d1 + 1e-16)
    r = jnp.concatenate(
        [jnp.broadcast_to(r0[:, None], (NBLK, C)),
         jnp.broadcast_to(r1[:, None], (NBLK, C))], axis=1)
    o_ref[...] = acc * r + skip_ref[...]


def _sc_gather_body(q_hbm, kv_hbm, src_hbm, dst_hbm, qd_hbm, kvs_hbm,
                    idxs_v, idxd_v, qrows_v, kvrows_v, sem1, sem2):
    wid = lax.axis_index("s") * NC + lax.axis_index("c")
    base = wid * EPW

    def win(g, carry):
        b = pl.multiple_of(base + g * GW, 8)
        pltpu.sync_copy(src_hbm.at[pl.ds(b, GW)], idxs_v)
        pltpu.sync_copy(dst_hbm.at[pl.ds(b, GW)], idxd_v)
        c1 = pltpu.async_copy(q_hbm.at[idxd_v], qrows_v, sem1)
        c2 = pltpu.async_copy(kv_hbm.at[idxs_v], kvrows_v, sem2)
        c1.wait()
        c2.wait()
        pltpu.sync_copy(qrows_v, qd_hbm.at[pl.ds(b, GW)])
        pltpu.sync_copy(kvrows_v, kvs_hbm.at[pl.ds(b, GW)])
        return carry

    lax.fori_loop(0, NWIN, win, 0)


def _sc_gather(q, kvp, src, dst):
    return pl.kernel(
        _sc_gather_body,
        out_type=[
            jax.ShapeDtypeStruct((E, HC), jnp.float32),
            jax.ShapeDtypeStruct((E, HC), jnp.float32),
        ],
        mesh=plsc.VectorSubcoreMesh(core_axis_name="c", subcore_axis_name="s"),
        scratch_types=[
            pltpu.VMEM((GW,), jnp.int32),
            pltpu.VMEM((GW,), jnp.int32),
            pltpu.VMEM((GW, HC), jnp.float32),
            pltpu.VMEM((GW, HC), jnp.float32),
            pltpu.SemaphoreType.DMA,
            pltpu.SemaphoreType.DMA,
        ],
    )(q, kvp, src, dst)


def _sc_scatter_body(wv_hbm, ex0_hbm, ex1_hbm, dst_hbm,
                     part_hbm, den_hbm,
                     idxd_v, wrows_v, ex0_v, ex1_v,
                     idxd2_v, wrows2_v, ex02_v, ex12_v,
                     zr_v, zd_v, out_sh, den0_sh, den1_sh,
                     smi0, smw0, sm00, sm10, smi1, smw1, sm01, sm11):
    cid = lax.axis_index("c")
    sid = lax.axis_index("s")
    wid = sid * NC + cid
    base = wid * EPW

    zv16 = jnp.zeros((16,), jnp.float32)
    for r in range(zr_v.shape[0]):
        for cc in range(HC // 16):
            zr_v[r, pl.ds(cc * 16, 16)] = zv16
    for cc in range(NPT // 16):
        zd_v[pl.ds(cc * 16, 16)] = zv16
    nzr = zr_v.shape[0]
    for rep in range(NPT // nzr):
        pltpu.sync_copy(zr_v, out_sh.at[pl.ds(sid * NPT + rep * nzr, nzr)])
    pltpu.sync_copy(zd_v, den0_sh.at[pl.ds(sid * NPT, NPT)])
    pltpu.sync_copy(zd_v, den1_sh.at[pl.ds(sid * NPT, NPT)])
    plsc.subcore_barrier()

    def load(g, idxd, wr, e0, e1, si, sw, s0, s1):
        b = pl.multiple_of(base + g * GW, 8)
        pltpu.async_copy(dst_hbm.at[pl.ds(b, GW)], idxd.at[0], si)
        pltpu.async_copy(wv_hbm.at[pl.ds(b, GW)], wr, sw)
        pltpu.async_copy(ex0_hbm.at[pl.ds(b, GW)], e0, s0)
        pltpu.async_copy(ex1_hbm.at[pl.ds(b, GW)], e1, s1)

    def wait_l(g, idxd, wr, e0, e1, si, sw, s0, s1):
        b = pl.multiple_of(base + g * GW, 8)
        pltpu.make_async_copy(dst_hbm.at[pl.ds(b, GW)], idxd.at[0], si).wait()
        pltpu.make_async_copy(wv_hbm.at[pl.ds(b, GW)], wr, sw).wait()
        pltpu.make_async_copy(ex0_hbm.at[pl.ds(b, GW)], e0, s0).wait()
        pltpu.make_async_copy(ex1_hbm.at[pl.ds(b, GW)], e1, s1).wait()

    def scat(idxd, wr, e0, e1):
        pltpu.sync_copy(wr, out_sh.at[idxd.at[0]], add=True)
        pltpu.sync_copy(e0, den0_sh.at[idxd.at[0]], add=True)
        pltpu.sync_copy(e1, den1_sh.at[idxd.at[0]], add=True)

    set0 = (idxd_v, wrows_v, ex0_v, ex1_v, smi0, smw0, sm00, sm10)
    set1 = (idxd2_v, wrows2_v, ex02_v, ex12_v, smi1, smw1, sm01, sm11)

    load(0, *set0)

    def pair(h, carry):
        g0 = 2 * h
        g1 = g0 + 1
        wait_l(g0, *set0)
        load(g1, *set1)
        scat(idxd_v, wrows_v, ex0_v, ex1_v)
        wait_l(g1, *set1)

        @pl.when(g0 + 2 < NWIN)
        def _():
            load(g0 + 2, *set0)

        scat(idxd2_v, wrows2_v, ex02_v, ex12_v)
        return carry

    lax.fori_loop(0, NWIN // 2, pair, 0)
    wait_l(NWIN - 1, *set0)
    scat(idxd_v, wrows_v, ex0_v, ex1_v)
    plsc.subcore_barrier()

    pltpu.sync_copy(out_sh.at[pl.ds(sid * NPT, NPT)],
                    part_hbm.at[cid, pl.ds(sid * NPT, NPT)])
    pltpu.sync_copy(den0_sh.at[pl.ds(sid * NPT, NPT)],
                    den_hbm.at[cid, 0, pl.ds(sid * NPT, NPT)])
    pltpu.sync_copy(den1_sh.at[pl.ds(sid * NPT, NPT)],
                    den_hbm.at[cid, 1, pl.ds(sid * NPT, NPT)])


def _sc_scatter(wv, ex0, ex1, dst):
    return pl.kernel(
        _sc_scatter_body,
        out_type=[
            jax.ShapeDtypeStruct((NC, NPAD, HC), jnp.float32),
            jax.ShapeDtypeStruct((NC, 2, NPAD), jnp.float32),
        ],
        mesh=plsc.VectorSubcoreMesh(core_axis_name="c", subcore_axis_name="s"),
        scratch_types=[
            pltpu.VMEM((1, GW), jnp.int32),
            pltpu.VMEM((GW, HC), jnp.float32),
            pltpu.VMEM((GW,), jnp.float32),
            pltpu.VMEM((GW,), jnp.float32),
            pltpu.VMEM((1, GW), jnp.int32),
            pltpu.VMEM((GW, HC), jnp.float32),
            pltpu.VMEM((GW,), jnp.float32),
            pltpu.VMEM((GW,), jnp.float32),
            pltpu.VMEM((64, HC), jnp.float32),
            pltpu.VMEM((NPT,), jnp.float32),
            pltpu.VMEM_SHARED((NPAD, HC), jnp.float32),
            pltpu.VMEM_SHARED((NPAD,), jnp.float32),
            pltpu.VMEM_SHARED((NPAD,), jnp.float32),
        ] + [pltpu.SemaphoreType.DMA] * 8,
    )(wv, ex0, ex1, dst)


def kernel(x, edge_index, edge_time, msg, time_w, time_b, Wq, bq, Wk, bk, Wv, bv, We, Wskip, bskip):
    Wall = jnp.concatenate([Wq.T, Wk.T, Wv.T, Wskip.T], axis=1)
    ball = jnp.concatenate([bq, bk, bv, bskip])[None, :]

    q, kvp, skip = pl.pallas_call(
        _qkvg_body,
        grid=(N // 1000,),
        in_specs=[
            pl.BlockSpec((1000, D), lambda i: (i, 0)),
            pl.BlockSpec((D, 4 * HC), lambda i: (0, 0)),
            pl.BlockSpec((1, 4 * HC), lambda i: (0, 0)),
        ],
        out_specs=[
            pl.BlockSpec((1000, HC), lambda i: (i, 0)),
            pl.BlockSpec((1000, HC), lambda i: (i, 0)),
            pl.BlockSpec((1000, HC), lambda i: (i, 0)),
        ],
        out_shape=[
            jax.ShapeDtypeStruct((N, HC), jnp.float32),
            jax.ShapeDtypeStruct((N, HC), jnp.float32),
            jax.ShapeDtypeStruct((N, HC), jnp.float32),
        ],
    )(x, Wall, ball)

    wt = jnp.pad(time_w, (0, HC - TD))[None, :]
    bt = jnp.pad(time_b, (0, HC - TD))[None, :]
    WeTp = jnp.pad(We.T, ((0, HC - TD), (0, 0))).astype(jnp.bfloat16)
    t2 = edge_time[:, None]

    e = pl.pallas_call(
        _time_e_body,
        grid=(E // EBLK,),
        in_specs=[
            pl.BlockSpec((EBLK, 1), lambda i: (i, 0)),
            pl.BlockSpec((1, HC), lambda i: (0, 0)),
            pl.BlockSpec((1, HC), lambda i: (0, 0)),
            pl.BlockSpec((HC, HC), lambda i: (0, 0)),
        ],
        out_specs=pl.BlockSpec((EBLK, HC), lambda i: (i, 0)),
        out_shape=jax.ShapeDtypeStruct((E, HC), jnp.float32),
    )(t2, wt, bt, WeTp)

    src = edge_index[0]
    dst = edge_index[1]
    qd, kvs = _sc_gather(q, kvp, src, dst)

    ex0, ex1, wv = pl.pallas_call(
        _alpha_body,
        grid=(E // EBLK,),
        in_specs=[
            pl.BlockSpec((EBLK, HC), lambda i: (i, 0)),
            pl.BlockSpec((EBLK, HC), lambda i: (i, 0)),
            pl.BlockSpec((EBLK, HC), lambda i: (i, 0)),
        ],
        out_specs=[
            pl.BlockSpec((1, EBLK), lambda i: (0, i)),
            pl.BlockSpec((1, EBLK), lambda i: (0, i)),
            pl.BlockSpec((EBLK, HC), lambda i: (i, 0)),
        ],
        out_shape=[
            jax.ShapeDtypeStruct((1, E), jnp.float32),
            jax.ShapeDtypeStruct((1, E), jnp.float32),
            jax.ShapeDtypeStruct((E, HC), jnp.float32),
        ],
    )(qd, kvs, e)

    part, den = _sc_scatter(wv, ex0.reshape(E), ex1.reshape(E), dst)

    skip_pad = jnp.pad(skip, ((0, NPAD - N), (0, 0)))
    out_pad = pl.pallas_call(
        _final_body,
        grid=(NPAD // NBLK,),
        in_specs=[
            pl.BlockSpec((NC, NBLK, HC), lambda i: (0, i, 0)),
            pl.BlockSpec((NC, 2, NBLK), lambda i: (0, 0, i)),
            pl.BlockSpec((NBLK, HC), lambda i: (i, 0)),
        ],
        out_specs=pl.BlockSpec((NBLK, HC), lambda i: (i, 0)),
        out_shape=jax.ShapeDtypeStruct((NPAD, HC), jnp.float32),
    )(part, den, skip_pad)
    return out_pad[:N]


# speedup vs baseline: 1.6501x; 1.3375x over previous
import functools

import jax
import jax.numpy as jnp
from jax import lax
from jax.experimental import pallas as pl
from jax.experimental.pallas import tpu as pltpu
from jax.experimental.pallas import tpu_sc as plsc

N = 10000
E = 320000
D = 128
H = 2
C = 64
HC = H * C
TD = 100

NPAD = 10240
NBLK = 1024
EBLK = 2560

_SC_INFO = plsc.get_sparse_core_info()
NC = _SC_INFO.num_cores
NS = _SC_INFO.num_subcores
NW = NC * NS
EPW = E // NW
GW = 80
NWIN = EPW // GW
NPT = NPAD // NS


def _qkvg_body(x_ref, w_ref, b_ref, q_ref, kvp_ref, g_ref):
    o = (jnp.dot(x_ref[...], w_ref[...], preferred_element_type=jnp.float32)
         + b_ref[...])
    u32 = jnp.uint32
    kb = lax.bitcast_convert_type(
        o[:, HC:2 * HC].astype(jnp.bfloat16).astype(jnp.float32), u32)
    vb = lax.bitcast_convert_type(
        o[:, 2 * HC:3 * HC].astype(jnp.bfloat16).astype(jnp.float32), u32)
    kvp = (kb & jnp.uint32(0xFFFF0000)) | (vb >> 16)
    q_ref[...] = o[:, :HC]
    kvp_ref[...] = lax.bitcast_convert_type(kvp, jnp.float32)
    g_ref[...] = o[:, 3 * HC:]


def _time_e_body(t_ref, wt_ref, bt_ref, we_ref, e_ref):
    enc = jnp.cos(t_ref[...] * wt_ref[...] + bt_ref[...])
    e_ref[...] = jnp.dot(enc.astype(jnp.bfloat16), we_ref[...],
                         preferred_element_type=jnp.float32)


def _unpack(u, hi):
    h = (u & jnp.uint32(0xFFFF0000)) if hi else (u << 16)
    return lax.bitcast_convert_type(h, jnp.float32)


def _alpha_body(qd_ref, kvs_ref, e_ref, ex0_ref, ex1_ref, wv_ref):
    eb = e_ref[...]
    kv_u = lax.bitcast_convert_type(kvs_ref[...], jnp.uint32)
    ks = _unpack(kv_u, True)
    vs = _unpack(kv_u, False)
    a = qd_ref[...] * (ks + eb)
    h0 = jnp.sum(a[:, :C], axis=1) * (1.0 / 8.0)
    h1 = jnp.sum(a[:, C:], axis=1) * (1.0 / 8.0)
    ex0 = jnp.exp(h0)
    ex1 = jnp.exp(h1)
    ex0_ref[...] = ex0[None, :]
    ex1_ref[...] = ex1[None, :]
    w = jnp.concatenate(
        [jnp.broadcast_to(ex0[:, None], (EBLK, C)),
         jnp.broadcast_to(ex1[:, None], (EBLK, C))], axis=1)
    wv_ref[...] = (vs + eb) * w


def _final_body(p_ref, den_ref, skip_ref, o_ref):
    acc = p_ref[0] + p_ref[1]
    d0 = den_ref[0, 0] + den_ref[1, 0]
    d1 = den_ref[0, 1] + den_ref[1, 1]
    r0 = 1.0 / (d0 + 1e-16)
    r1 = 1.0 / (d1 + 1e-16)
    r = jnp.concatenate(
        [jnp.broadcast_to(r0[:, None], (NBLK, C)),
         jnp.broadcast_to(r1[:, None], (NBLK, C))], axis=1)
    o_ref[...] = acc * r + skip_ref[...]


def _sc_gather_body(q_hbm, kv_hbm, src_hbm, dst_hbm, qd_hbm, kvs_hbm,
                    idxs_v, idxd_v, qrows_v, kvrows_v, sem1, sem2):
    wid = lax.axis_index("s") * NC + lax.axis_index("c")
    base = wid * EPW

    def win(g, carry):
        b = pl.multiple_of(base + g * GW, 8)
        pltpu.sync_copy(src_hbm.at[pl.ds(b, GW)], idxs_v)
        pltpu.sync_copy(dst_hbm.at[pl.ds(b, GW)], idxd_v)
        c1 = pltpu.async_copy(q_hbm.at[idxd_v], qrows_v, sem1)
        c2 = pltpu.async_copy(kv_hbm.at[idxs_v], kvrows_v, sem2)
        c1.wait()
        c2.wait()
        pltpu.sync_copy(qrows_v, qd_hbm.at[pl.ds(b, GW)])
        pltpu.sync_copy(kvrows_v, kvs_hbm.at[pl.ds(b, GW)])
        return carry

    lax.fori_loop(0, NWIN, win, 0)


def _sc_gather(q, kvp, src, dst):
    return pl.kernel(
        _sc_gather_body,
        out_type=[
            jax.ShapeDtypeStruct((E, HC), jnp.float32),
            jax.ShapeDtypeStruct((E, HC), jnp.float32),
        ],
        mesh=plsc.VectorSubcoreMesh(core_axis_name="c", subcore_axis_name="s"),
        scratch_types=[
            pltpu.VMEM((GW,), jnp.int32),
            pltpu.VMEM((GW,), jnp.int32),
            pltpu.VMEM((GW, HC), jnp.float32),
            pltpu.VMEM((GW, HC), jnp.float32),
            pltpu.SemaphoreType.DMA,
            pltpu.SemaphoreType.DMA,
        ],
    )(q, kvp, src, dst)


def _sc_scatter_body(wv_hbm, ex0_hbm, ex1_hbm, dst_hbm,
                     part_hbm, den_hbm,
                     idxd_v, wrows_v, ex0_v, ex1_v,
                     idxd2_v, wrows2_v, ex02_v, ex12_v,
                     zr_v, zd_v, out_sh, den0_sh, den1_sh,
                     smi0, smw0, sm00, sm10, smi1, smw1, sm01, sm11):
    cid = lax.axis_index("c")
    sid = lax.axis_index("s")
    wid = sid * NC + cid
    base = wid * EPW

    zv16 = jnp.zeros((16,), jnp.float32)
    for r in range(zr_v.shape[0]):
        for cc in range(HC // 16):
            zr_v[r, pl.ds(cc * 16, 16)] = zv16
    for cc in range(NPT // 16):
        zd_v[pl.ds(cc * 16, 16)] = zv16
    nzr = zr_v.shape[0]
    for rep in range(NPT // nzr):
        pltpu.sync_copy(zr_v, out_sh.at[pl.ds(sid * NPT + rep * nzr, nzr)])
    pltpu.sync_copy(zd_v, den0_sh.at[pl.ds(sid * NPT, NPT)])
    pltpu.sync_copy(zd_v, den1_sh.at[pl.ds(sid * NPT, NPT)])
    plsc.subcore_barrier()

    def load(g, idxd, wr, e0, e1, si, sw, s0, s1):
        b = pl.multiple_of(base + g * GW, 8)
        pltpu.async_copy(dst_hbm.at[pl.ds(b, GW)], idxd.at[0], si)
        pltpu.async_copy(wv_hbm.at[pl.ds(b, GW)], wr, sw)
        pltpu.async_copy(ex0_hbm.at[pl.ds(b, GW)], e0, s0)
        pltpu.async_copy(ex1_hbm.at[pl.ds(b, GW)], e1, s1)

    def wait_l(g, idxd, wr, e0, e1, si, sw, s0, s1):
        b = pl.multiple_of(base + g * GW, 8)
        pltpu.make_async_copy(dst_hbm.at[pl.ds(b, GW)], idxd.at[0], si).wait()
        pltpu.make_async_copy(wv_hbm.at[pl.ds(b, GW)], wr, sw).wait()
        pltpu.make_async_copy(ex0_hbm.at[pl.ds(b, GW)], e0, s0).wait()
        pltpu.make_async_copy(ex1_hbm.at[pl.ds(b, GW)], e1, s1).wait()

    def scat(idxd, wr, e0, e1):
        pltpu.sync_copy(wr, out_sh.at[idxd.at[0]], add=True)
        pltpu.sync_copy(e0, den0_sh.at[idxd.at[0]], add=True)
        pltpu.sync_copy(e1, den1_sh.at[idxd.at[0]], add=True)

    set0 = (idxd_v, wrows_v, ex0_v, ex1_v, smi0, smw0, sm00, sm10)
    set1 = (idxd2_v, wrows2_v, ex02_v, ex12_v, smi1, smw1, sm01, sm11)

    load(0, *set0)

    def pair(h, carry):
        g0 = 2 * h
        g1 = g0 + 1
        wait_l(g0, *set0)
        load(g1, *set1)
        scat(idxd_v, wrows_v, ex0_v, ex1_v)
        wait_l(g1, *set1)

        @pl.when(g0 + 2 < NWIN)
        def _():
            load(g0 + 2, *set0)

        scat(idxd2_v, wrows2_v, ex02_v, ex12_v)
        return carry

    lax.fori_loop(0, NWIN // 2, pair, 0)
    wait_l(NWIN - 1, *set0)
    scat(idxd_v, wrows_v, ex0_v, ex1_v)
    plsc.subcore_barrier()

    pltpu.sync_copy(out_sh.at[pl.ds(sid * NPT, NPT)],
                    part_hbm.at[cid, pl.ds(sid * NPT, NPT)])
    pltpu.sync_copy(den0_sh.at[pl.ds(sid * NPT, NPT)],
                    den_hbm.at[cid, 0, pl.ds(sid * NPT, NPT)])
    pltpu.sync_copy(den1_sh.at[pl.ds(sid * NPT, NPT)],
                    den_hbm.at[cid, 1, pl.ds(sid * NPT, NPT)])


def _sc_scatter(wv, ex0, ex1, dst):
    return pl.kernel(
        _sc_scatter_body,
        out_type=[
            jax.ShapeDtypeStruct((NC, NPAD, HC), jnp.float32),
            jax.ShapeDtypeStruct((NC, 2, NPAD), jnp.float32),
        ],
        mesh=plsc.VectorSubcoreMesh(core_axis_name="c", subcore_axis_name="s"),
        scratch_types=[
            pltpu.VMEM((1, GW), jnp.int32),
            pltpu.VMEM((GW, HC), jnp.float32),
            pltpu.VMEM((GW,), jnp.float32),
            pltpu.VMEM((GW,), jnp.float32),
            pltpu.VMEM((1, GW), jnp.int32),
            pltpu.VMEM((GW, HC), jnp.float32),
            pltpu.VMEM((GW,), jnp.float32),
            pltpu.VMEM((GW,), jnp.float32),
            pltpu.VMEM((64, HC), jnp.float32),
            pltpu.VMEM((NPT,), jnp.float32),
            pltpu.VMEM_SHARED((NPAD, HC), jnp.float32),
            pltpu.VMEM_SHARED((NPAD,), jnp.float32),
            pltpu.VMEM_SHARED((NPAD,), jnp.float32),
        ] + [pltpu.SemaphoreType.DMA] * 8,
    )(wv, ex0, ex1, dst)


def kernel(x, edge_index, edge_time, msg, time_w, time_b, Wq, bq, Wk, bk, Wv, bv, We, Wskip, bskip):
    Wall = jnp.concatenate([Wq.T, Wk.T, Wv.T, Wskip.T], axis=1)
    ball = jnp.concatenate([bq, bk, bv, bskip])[None, :]

    q, kvp, skip = pl.pallas_call(
        _qkvg_body,
        grid=(N // 1000,),
        in_specs=[
            pl.BlockSpec((1000, D), lambda i: (i, 0)),
            pl.BlockSpec((D, 4 * HC), lambda i: (0, 0)),
            pl.BlockSpec((1, 4 * HC), lambda i: (0, 0)),
        ],
        out_specs=[
            pl.BlockSpec((1000, HC), lambda i: (i, 0)),
            pl.BlockSpec((1000, HC), lambda i: (i, 0)),
            pl.BlockSpec((1000, HC), lambda i: (i, 0)),
        ],
        out_shape=[
            jax.ShapeDtypeStruct((N, HC), jnp.float32),
            jax.ShapeDtypeStruct((N, HC), jnp.float32),
            jax.ShapeDtypeStruct((N, HC), jnp.float32),
        ],
    )(x, Wall, ball)

    wt = jnp.pad(time_w, (0, HC - TD))[None, :]
    bt = jnp.pad(time_b, (0, HC - TD))[None, :]
    WeTp = jnp.pad(We.T, ((0, HC - TD), (0, 0))).astype(jnp.bfloat16)
    t2 = edge_time[:, None]

    e = pl.pallas_call(
        _time_e_body,
        grid=(E // EBLK,),
        in_specs=[
            pl.BlockSpec((EBLK, 1), lambda i: (i, 0)),
            pl.BlockSpec((1, HC), lambda i: (0, 0)),
            pl.BlockSpec((1, HC), lambda i: (0, 0)),
            pl.BlockSpec((HC, HC), lambda i: (0, 0)),
        ],
        out_specs=pl.BlockSpec((EBLK, HC), lambda i: (i, 0)),
        out_shape=jax.ShapeDtypeStruct((E, HC), jnp.float32),
    )(t2, wt, bt, WeTp)

    src = edge_index[0]
    dst = edge_index[1]
    qd, kvs = _sc_gather(q, kvp, src, dst)

    ex0, ex1, wv = pl.pallas_call(
        _alpha_body,
        grid=(E // EBLK,),
        in_specs=[
            pl.BlockSpec((EBLK, HC), lambda i: (i, 0)),
            pl.BlockSpec((EBLK, HC), lambda i: (i, 0)),
            pl.BlockSpec((EBLK, HC), lambda i: (i, 0)),
        ],
        out_specs=[
            pl.BlockSpec((1, EBLK), lambda i: (0, i)),
            pl.BlockSpec((1, EBLK), lambda i: (0, i)),
            pl.BlockSpec((EBLK, HC), lambda i: (i, 0)),
        ],
        out_shape=[
            jax.ShapeDtypeStruct((1, E), jnp.float32),
            jax.ShapeDtypeStruct((1, E), jnp.float32),
            jax.ShapeDtypeStruct((E, HC), jnp.float32),
        ],
    )(qd, kvs, e)

    part, den = _sc_scatter(wv, ex0.reshape(E), ex1.reshape(E), dst)

    skip_pad = jnp.pad(skip, ((0, NPAD - N), (0, 0)))
    out_pad = pl.pallas_call(
        _final_body,
        grid=(NPAD // NBLK,),
        in_specs=[
            pl.BlockSpec((NC, NBLK, HC), lambda i: (0, i, 0)),
            pl.BlockSpec((NC, 2, NBLK), lambda i: (0, 0, i)),
            pl.BlockSpec((NBLK, HC), lambda i: (i, 0)),
        ],
        out_specs=pl.BlockSpec((NBLK, HC), lambda i: (i, 0)),
        out_shape=jax.ShapeDtypeStruct((NPAD, HC), jnp.float32),
    )(part, den, skip_pad)
    return out_pad[:N]


# speedup vs baseline: 1.6847x; 1.0210x over previous
import functools

import jax
import jax.numpy as jnp
from jax import lax
from jax.experimental import pallas as pl
from jax.experimental.pallas import tpu as pltpu
from jax.experimental.pallas import tpu_sc as plsc

N = 10000
E = 320000
D = 128
H = 2
C = 64
HC = H * C
TD = 100

NPAD = 10240
NBLK = 1024
EBLK = 6400

_SC_INFO = plsc.get_sparse_core_info()
NC = _SC_INFO.num_cores
NS = _SC_INFO.num_subcores
NW = NC * NS
EPW = E // NW
GW = 80
NWIN = EPW // GW
NPT = NPAD // NS


def _qkvg_body(x_ref, w_ref, b_ref, q_ref, kvp_ref, g_ref):
    o = (jnp.dot(x_ref[...], w_ref[...], preferred_element_type=jnp.float32)
         + b_ref[...])
    u32 = jnp.uint32
    kb = lax.bitcast_convert_type(
        o[:, HC:2 * HC].astype(jnp.bfloat16).astype(jnp.float32), u32)
    vb = lax.bitcast_convert_type(
        o[:, 2 * HC:3 * HC].astype(jnp.bfloat16).astype(jnp.float32), u32)
    kvp = (kb & jnp.uint32(0xFFFF0000)) | (vb >> 16)
    q_ref[...] = o[:, :HC]
    kvp_ref[...] = lax.bitcast_convert_type(kvp, jnp.float32)
    g_ref[...] = o[:, 3 * HC:]


def _time_e_body(t_ref, wt_ref, bt_ref, we_ref, e_ref):
    enc = jnp.cos(t_ref[...] * wt_ref[...] + bt_ref[...])
    e_ref[...] = jnp.dot(enc.astype(jnp.bfloat16), we_ref[...],
                         preferred_element_type=jnp.float32)


def _unpack(u, hi):
    h = (u & jnp.uint32(0xFFFF0000)) if hi else (u << 16)
    return lax.bitcast_convert_type(h, jnp.float32)


def _alpha_body(qd_ref, kvs_ref, e_ref, ex0_ref, ex1_ref, wv_ref):
    eb = e_ref[...]
    kv_u = lax.bitcast_convert_type(kvs_ref[...], jnp.uint32)
    ks = _unpack(kv_u, True)
    vs = _unpack(kv_u, False)
    a = qd_ref[...] * (ks + eb)
    h0 = jnp.sum(a[:, :C], axis=1) * (1.0 / 8.0)
    h1 = jnp.sum(a[:, C:], axis=1) * (1.0 / 8.0)
    ex0 = jnp.exp(h0)
    ex1 = jnp.exp(h1)
    ex0_ref[...] = ex0[None, :]
    ex1_ref[...] = ex1[None, :]
    w = jnp.concatenate(
        [jnp.broadcast_to(ex0[:, None], (EBLK, C)),
         jnp.broadcast_to(ex1[:, None], (EBLK, C))], axis=1)
    wv_ref[...] = (vs + eb) * w


def _final_body(p_ref, den_ref, skip_ref, o_ref):
    acc = p_ref[0] + p_ref[1]
    d0 = den_ref[0, 0] + den_ref[1, 0]
    d1 = den_ref[0, 1] + den_ref[1, 1]
    r0 = 1.0 / (d0 + 1e-16)
    r1 = 1.0 / (d1 + 1e-16)
    r = jnp.concatenate(
        [jnp.broadcast_to(r0[:, None], (NBLK, C)),
         jnp.broadcast_to(r1[:, None], (NBLK, C))], axis=1)
    o_ref[...] = acc * r + skip_ref[...]


def _sc_gather_body(q_hbm, kv_hbm, src_hbm, dst_hbm, qd_hbm, kvs_hbm,
                    idxs_v, idxd_v, qrows_v, kvrows_v, sem1, sem2):
    wid = lax.axis_index("s") * NC + lax.axis_index("c")
    base = wid * EPW

    def win(g, carry):
        b = pl.multiple_of(base + g * GW, 8)
        pltpu.sync_copy(src_hbm.at[pl.ds(b, GW)], idxs_v)
        pltpu.sync_copy(dst_hbm.at[pl.ds(b, GW)], idxd_v)
        c1 = pltpu.async_copy(q_hbm.at[idxd_v], qrows_v, sem1)
        c2 = pltpu.async_copy(kv_hbm.at[idxs_v], kvrows_v, sem2)
        c1.wait()
        c2.wait()
        pltpu.sync_copy(qrows_v, qd_hbm.at[pl.ds(b, GW)])
        pltpu.sync_copy(kvrows_v, kvs_hbm.at[pl.ds(b, GW)])
        return carry

    lax.fori_loop(0, NWIN, win, 0)


def _sc_gather(q, kvp, src, dst):
    return pl.kernel(
        _sc_gather_body,
        out_type=[
            jax.ShapeDtypeStruct((E, HC), jnp.float32),
            jax.ShapeDtypeStruct((E, HC), jnp.float32),
        ],
        mesh=plsc.VectorSubcoreMesh(core_axis_name="c", subcore_axis_name="s"),
        scratch_types=[
            pltpu.VMEM((GW,), jnp.int32),
            pltpu.VMEM((GW,), jnp.int32),
            pltpu.VMEM((GW, HC), jnp.float32),
            pltpu.VMEM((GW, HC), jnp.float32),
            pltpu.SemaphoreType.DMA,
            pltpu.SemaphoreType.DMA,
        ],
    )(q, kvp, src, dst)


def _sc_scatter_body(wv_hbm, ex0_hbm, ex1_hbm, dst_hbm,
                     part_hbm, den_hbm,
                     idxd_v, wrows_v, ex0_v, ex1_v,
                     idxd2_v, wrows2_v, ex02_v, ex12_v,
                     zr_v, zd_v, out_sh, den0_sh, den1_sh,
                     smi0, smw0, sm00, sm10, smi1, smw1, sm01, sm11):
    cid = lax.axis_index("c")
    sid = lax.axis_index("s")
    wid = sid * NC + cid
    base = wid * EPW

    zv16 = jnp.zeros((16,), jnp.float32)
    for r in range(zr_v.shape[0]):
        for cc in range(HC // 16):
            zr_v[r, pl.ds(cc * 16, 16)] = zv16
    for cc in range(NPT // 16):
        zd_v[pl.ds(cc * 16, 16)] = zv16
    nzr = zr_v.shape[0]
    for rep in range(NPT // nzr):
        pltpu.sync_copy(zr_v, out_sh.at[pl.ds(sid * NPT + rep * nzr, nzr)])
    pltpu.sync_copy(zd_v, den0_sh.at[pl.ds(sid * NPT, NPT)])
    pltpu.sync_copy(zd_v, den1_sh.at[pl.ds(sid * NPT, NPT)])
    plsc.subcore_barrier()

    def load(g, idxd, wr, e0, e1, si, sw, s0, s1):
        b = pl.multiple_of(base + g * GW, 8)
        pltpu.async_copy(dst_hbm.at[pl.ds(b, GW)], idxd.at[0], si)
        pltpu.async_copy(wv_hbm.at[pl.ds(b, GW)], wr, sw)
        pltpu.async_copy(ex0_hbm.at[pl.ds(b, GW)], e0, s0)
        pltpu.async_copy(ex1_hbm.at[pl.ds(b, GW)], e1, s1)

    def wait_l(g, idxd, wr, e0, e1, si, sw, s0, s1):
        b = pl.multiple_of(base + g * GW, 8)
        pltpu.make_async_copy(dst_hbm.at[pl.ds(b, GW)], idxd.at[0], si).wait()
        pltpu.make_async_copy(wv_hbm.at[pl.ds(b, GW)], wr, sw).wait()
        pltpu.make_async_copy(ex0_hbm.at[pl.ds(b, GW)], e0, s0).wait()
        pltpu.make_async_copy(ex1_hbm.at[pl.ds(b, GW)], e1, s1).wait()

    def scat(idxd, wr, e0, e1):
        pltpu.sync_copy(wr, out_sh.at[idxd.at[0]], add=True)
        pltpu.sync_copy(e0, den0_sh.at[idxd.at[0]], add=True)
        pltpu.sync_copy(e1, den1_sh.at[idxd.at[0]], add=True)

    set0 = (idxd_v, wrows_v, ex0_v, ex1_v, smi0, smw0, sm00, sm10)
    set1 = (idxd2_v, wrows2_v, ex02_v, ex12_v, smi1, smw1, sm01, sm11)

    load(0, *set0)

    def pair(h, carry):
        g0 = 2 * h
        g1 = g0 + 1
        wait_l(g0, *set0)
        load(g1, *set1)
        scat(idxd_v, wrows_v, ex0_v, ex1_v)
        wait_l(g1, *set1)

        @pl.when(g0 + 2 < NWIN)
        def _():
            load(g0 + 2, *set0)

        scat(idxd2_v, wrows2_v, ex02_v, ex12_v)
        return carry

    lax.fori_loop(0, NWIN // 2, pair, 0)
    wait_l(NWIN - 1, *set0)
    scat(idxd_v, wrows_v, ex0_v, ex1_v)
    plsc.subcore_barrier()

    pltpu.sync_copy(out_sh.at[pl.ds(sid * NPT, NPT)],
                    part_hbm.at[cid, pl.ds(sid * NPT, NPT)])
    pltpu.sync_copy(den0_sh.at[pl.ds(sid * NPT, NPT)],
                    den_hbm.at[cid, 0, pl.ds(sid * NPT, NPT)])
    pltpu.sync_copy(den1_sh.at[pl.ds(sid * NPT, NPT)],
                    den_hbm.at[cid, 1, pl.ds(sid * NPT, NPT)])


def _sc_scatter(wv, ex0, ex1, dst):
    return pl.kernel(
        _sc_scatter_body,
        out_type=[
            jax.ShapeDtypeStruct((NC, NPAD, HC), jnp.float32),
            jax.ShapeDtypeStruct((NC, 2, NPAD), jnp.float32),
        ],
        mesh=plsc.VectorSubcoreMesh(core_axis_name="c", subcore_axis_name="s"),
        scratch_types=[
            pltpu.VMEM((1, GW), jnp.int32),
            pltpu.VMEM((GW, HC), jnp.float32),
            pltpu.VMEM((GW,), jnp.float32),
            pltpu.VMEM((GW,), jnp.float32),
            pltpu.VMEM((1, GW), jnp.int32),
            pltpu.VMEM((GW, HC), jnp.float32),
            pltpu.VMEM((GW,), jnp.float32),
            pltpu.VMEM((GW,), jnp.float32),
            pltpu.VMEM((64, HC), jnp.float32),
            pltpu.VMEM((NPT,), jnp.float32),
            pltpu.VMEM_SHARED((NPAD, HC), jnp.float32),
            pltpu.VMEM_SHARED((NPAD,), jnp.float32),
            pltpu.VMEM_SHARED((NPAD,), jnp.float32),
        ] + [pltpu.SemaphoreType.DMA] * 8,
    )(wv, ex0, ex1, dst)


def kernel(x, edge_index, edge_time, msg, time_w, time_b, Wq, bq, Wk, bk, Wv, bv, We, Wskip, bskip):
    Wall = jnp.concatenate([Wq.T, Wk.T, Wv.T, Wskip.T], axis=1)
    ball = jnp.concatenate([bq, bk, bv, bskip])[None, :]

    q, kvp, skip = pl.pallas_call(
        _qkvg_body,
        grid=(N // 1000,),
        in_specs=[
            pl.BlockSpec((1000, D), lambda i: (i, 0)),
            pl.BlockSpec((D, 4 * HC), lambda i: (0, 0)),
            pl.BlockSpec((1, 4 * HC), lambda i: (0, 0)),
        ],
        out_specs=[
            pl.BlockSpec((1000, HC), lambda i: (i, 0)),
            pl.BlockSpec((1000, HC), lambda i: (i, 0)),
            pl.BlockSpec((1000, HC), lambda i: (i, 0)),
        ],
        out_shape=[
            jax.ShapeDtypeStruct((N, HC), jnp.float32),
            jax.ShapeDtypeStruct((N, HC), jnp.float32),
            jax.ShapeDtypeStruct((N, HC), jnp.float32),
        ],
    )(x, Wall, ball)

    wt = jnp.pad(time_w, (0, HC - TD))[None, :]
    bt = jnp.pad(time_b, (0, HC - TD))[None, :]
    WeTp = jnp.pad(We.T, ((0, HC - TD), (0, 0))).astype(jnp.bfloat16)
    t2 = edge_time[:, None]

    e = pl.pallas_call(
        _time_e_body,
        grid=(E // EBLK,),
        in_specs=[
            pl.BlockSpec((EBLK, 1), lambda i: (i, 0)),
            pl.BlockSpec((1, HC), lambda i: (0, 0)),
            pl.BlockSpec((1, HC), lambda i: (0, 0)),
            pl.BlockSpec((HC, HC), lambda i: (0, 0)),
        ],
        out_specs=pl.BlockSpec((EBLK, HC), lambda i: (i, 0)),
        out_shape=jax.ShapeDtypeStruct((E, HC), jnp.float32),
    )(t2, wt, bt, WeTp)

    src = edge_index[0]
    dst = edge_index[1]
    qd, kvs = _sc_gather(q, kvp, src, dst)

    ex0, ex1, wv = pl.pallas_call(
        _alpha_body,
        grid=(E // EBLK,),
        in_specs=[
            pl.BlockSpec((EBLK, HC), lambda i: (i, 0)),
            pl.BlockSpec((EBLK, HC), lambda i: (i, 0)),
            pl.BlockSpec((EBLK, HC), lambda i: (i, 0)),
        ],
        out_specs=[
            pl.BlockSpec((1, EBLK), lambda i: (0, i)),
            pl.BlockSpec((1, EBLK), lambda i: (0, i)),
            pl.BlockSpec((EBLK, HC), lambda i: (i, 0)),
        ],
        out_shape=[
            jax.ShapeDtypeStruct((1, E), jnp.float32),
            jax.ShapeDtypeStruct((1, E), jnp.float32),
            jax.ShapeDtypeStruct((E, HC), jnp.float32),
        ],
    )(qd, kvs, e)

    part, den = _sc_scatter(wv, ex0.reshape(E), ex1.reshape(E), dst)

    skip_pad = jnp.pad(skip, ((0, NPAD - N), (0, 0)))
    out_pad = pl.pallas_call(
        _final_body,
        grid=(NPAD // NBLK,),
        in_specs=[
            pl.BlockSpec((NC, NBLK, HC), lambda i: (0, i, 0)),
            pl.BlockSpec((NC, 2, NBLK), lambda i: (0, 0, i)),
            pl.BlockSpec((NBLK, HC), lambda i: (i, 0)),
        ],
        out_specs=pl.BlockSpec((NBLK, HC), lambda i: (i, 0)),
        out_shape=jax.ShapeDtypeStruct((NPAD, HC), jnp.float32),
    )(part, den, skip_pad)
    return out_pad[:N]


# speedup vs baseline: 1.8764x; 1.1138x over previous
import functools

import jax
import jax.numpy as jnp
from jax import lax
from jax.experimental import pallas as pl
from jax.experimental.pallas import tpu as pltpu
from jax.experimental.pallas import tpu_sc as plsc

N = 10000
E = 320000
D = 128
H = 2
C = 64
HC = H * C
TD = 100

NPAD = 10240
NBLK = 1024
EBLK = 6400

_SC_INFO = plsc.get_sparse_core_info()
NC = _SC_INFO.num_cores
NS = _SC_INFO.num_subcores
NW = NC * NS
EPW = E // NW
GW = 80
NWIN = EPW // GW
NPT = NPAD // NS


def _qkvg_body(x_ref, w_ref, b_ref, q_ref, kvp_ref, g_ref):
    o = (jnp.dot(x_ref[...], w_ref[...], preferred_element_type=jnp.float32)
         + b_ref[...])
    u32 = jnp.uint32
    kb = lax.bitcast_convert_type(
        o[:, HC:2 * HC].astype(jnp.bfloat16).astype(jnp.float32), u32)
    vb = lax.bitcast_convert_type(
        o[:, 2 * HC:3 * HC].astype(jnp.bfloat16).astype(jnp.float32), u32)
    kvp = (kb & jnp.uint32(0xFFFF0000)) | (vb >> 16)
    q_ref[...] = o[:, :HC]
    kvp_ref[...] = lax.bitcast_convert_type(kvp, jnp.float32)
    g_ref[...] = o[:, 3 * HC:]


def _time_e_body(t_ref, wt_ref, bt_ref, we_ref, e_ref):
    encT = jnp.cos(wt_ref[...] * t_ref[...] + bt_ref[...])
    enc = jnp.transpose(encT.astype(jnp.bfloat16))
    e_ref[...] = jnp.dot(enc, we_ref[...],
                         preferred_element_type=jnp.float32)


def _unpack(u, hi):
    h = (u & jnp.uint32(0xFFFF0000)) if hi else (u << 16)
    return lax.bitcast_convert_type(h, jnp.float32)


def _alpha_body(qd_ref, kvs_ref, e_ref, ex0_ref, ex1_ref, wv_ref):
    eb = e_ref[...]
    kv_u = lax.bitcast_convert_type(kvs_ref[...], jnp.uint32)
    ks = _unpack(kv_u, True)
    vs = _unpack(kv_u, False)
    a = qd_ref[...] * (ks + eb)
    h0 = jnp.sum(a[:, :C], axis=1) * (1.0 / 8.0)
    h1 = jnp.sum(a[:, C:], axis=1) * (1.0 / 8.0)
    ex0 = jnp.exp(h0)
    ex1 = jnp.exp(h1)
    ex0_ref[...] = ex0[None, :]
    ex1_ref[...] = ex1[None, :]
    w = jnp.concatenate(
        [jnp.broadcast_to(ex0[:, None], (EBLK, C)),
         jnp.broadcast_to(ex1[:, None], (EBLK, C))], axis=1)
    wv_ref[...] = (vs + eb) * w


def _final_body(p_ref, den_ref, skip_ref, o_ref):
    acc = p_ref[0] + p_ref[1]
    d0 = den_ref[0, 0] + den_ref[1, 0]
    d1 = den_ref[0, 1] + den_ref[1, 1]
    r0 = 1.0 / (d0 + 1e-16)
    r1 = 1.0 / (d1 + 1e-16)
    r = jnp.concatenate(
        [jnp.broadcast_to(r0[:, None], (NBLK, C)),
         jnp.broadcast_to(r1[:, None], (NBLK, C))], axis=1)
    o_ref[...] = acc * r + skip_ref[...]


def _sc_gather_body(q_hbm, kv_hbm, src_hbm, dst_hbm, qd_hbm, kvs_hbm,
                    idxs_v, idxd_v, qrows_v, kvrows_v, sem1, sem2):
    wid = lax.axis_index("s") * NC + lax.axis_index("c")
    base = wid * EPW

    def win(g, carry):
        b = pl.multiple_of(base + g * GW, 8)
        pltpu.sync_copy(src_hbm.at[pl.ds(b, GW)], idxs_v)
        pltpu.sync_copy(dst_hbm.at[pl.ds(b, GW)], idxd_v)
        c1 = pltpu.async_copy(q_hbm.at[idxd_v], qrows_v, sem1)
        c2 = pltpu.async_copy(kv_hbm.at[idxs_v], kvrows_v, sem2)
        c1.wait()
        c2.wait()
        pltpu.sync_copy(qrows_v, qd_hbm.at[pl.ds(b, GW)])
        pltpu.sync_copy(kvrows_v, kvs_hbm.at[pl.ds(b, GW)])
        return carry

    lax.fori_loop(0, NWIN, win, 0)


def _sc_gather(q, kvp, src, dst):
    return pl.kernel(
        _sc_gather_body,
        out_type=[
            jax.ShapeDtypeStruct((E, HC), jnp.float32),
            jax.ShapeDtypeStruct((E, HC), jnp.float32),
        ],
        mesh=plsc.VectorSubcoreMesh(core_axis_name="c", subcore_axis_name="s"),
        scratch_types=[
            pltpu.VMEM((GW,), jnp.int32),
            pltpu.VMEM((GW,), jnp.int32),
            pltpu.VMEM((GW, HC), jnp.float32),
            pltpu.VMEM((GW, HC), jnp.float32),
            pltpu.SemaphoreType.DMA,
            pltpu.SemaphoreType.DMA,
        ],
    )(q, kvp, src, dst)


def _sc_scatter_body(wv_hbm, ex0_hbm, ex1_hbm, dst_hbm,
                     part_hbm, den_hbm,
                     idxd_v, wrows_v, ex0_v, ex1_v,
                     idxd2_v, wrows2_v, ex02_v, ex12_v,
                     zr_v, zd_v, out_sh, den0_sh, den1_sh,
                     smi0, smw0, sm00, sm10, smi1, smw1, sm01, sm11):
    cid = lax.axis_index("c")
    sid = lax.axis_index("s")
    wid = sid * NC + cid
    base = wid * EPW

    zv16 = jnp.zeros((16,), jnp.float32)
    for r in range(zr_v.shape[0]):
        for cc in range(HC // 16):
            zr_v[r, pl.ds(cc * 16, 16)] = zv16
    for cc in range(NPT // 16):
        zd_v[pl.ds(cc * 16, 16)] = zv16
    nzr = zr_v.shape[0]
    for rep in range(NPT // nzr):
        pltpu.sync_copy(zr_v, out_sh.at[pl.ds(sid * NPT + rep * nzr, nzr)])
    pltpu.sync_copy(zd_v, den0_sh.at[pl.ds(sid * NPT, NPT)])
    pltpu.sync_copy(zd_v, den1_sh.at[pl.ds(sid * NPT, NPT)])
    plsc.subcore_barrier()

    def load(g, idxd, wr, e0, e1, si, sw, s0, s1):
        b = pl.multiple_of(base + g * GW, 8)
        pltpu.async_copy(dst_hbm.at[pl.ds(b, GW)], idxd.at[0], si)
        pltpu.async_copy(wv_hbm.at[pl.ds(b, GW)], wr, sw)
        pltpu.async_copy(ex0_hbm.at[pl.ds(b, GW)], e0, s0)
        pltpu.async_copy(ex1_hbm.at[pl.ds(b, GW)], e1, s1)

    def wait_l(g, idxd, wr, e0, e1, si, sw, s0, s1):
        b = pl.multiple_of(base + g * GW, 8)
        pltpu.make_async_copy(dst_hbm.at[pl.ds(b, GW)], idxd.at[0], si).wait()
        pltpu.make_async_copy(wv_hbm.at[pl.ds(b, GW)], wr, sw).wait()
        pltpu.make_async_copy(ex0_hbm.at[pl.ds(b, GW)], e0, s0).wait()
        pltpu.make_async_copy(ex1_hbm.at[pl.ds(b, GW)], e1, s1).wait()

    def scat(idxd, wr, e0, e1):
        pltpu.sync_copy(wr, out_sh.at[idxd.at[0]], add=True)
        pltpu.sync_copy(e0, den0_sh.at[idxd.at[0]], add=True)
        pltpu.sync_copy(e1, den1_sh.at[idxd.at[0]], add=True)

    set0 = (idxd_v, wrows_v, ex0_v, ex1_v, smi0, smw0, sm00, sm10)
    set1 = (idxd2_v, wrows2_v, ex02_v, ex12_v, smi1, smw1, sm01, sm11)

    load(0, *set0)

    def pair(h, carry):
        g0 = 2 * h
        g1 = g0 + 1
        wait_l(g0, *set0)
        load(g1, *set1)
        scat(idxd_v, wrows_v, ex0_v, ex1_v)
        wait_l(g1, *set1)

        @pl.when(g0 + 2 < NWIN)
        def _():
            load(g0 + 2, *set0)

        scat(idxd2_v, wrows2_v, ex02_v, ex12_v)
        return carry

    lax.fori_loop(0, NWIN // 2, pair, 0)
    wait_l(NWIN - 1, *set0)
    scat(idxd_v, wrows_v, ex0_v, ex1_v)
    plsc.subcore_barrier()

    pltpu.sync_copy(out_sh.at[pl.ds(sid * NPT, NPT)],
                    part_hbm.at[cid, pl.ds(sid * NPT, NPT)])
    pltpu.sync_copy(den0_sh.at[pl.ds(sid * NPT, NPT)],
                    den_hbm.at[cid, 0, pl.ds(sid * NPT, NPT)])
    pltpu.sync_copy(den1_sh.at[pl.ds(sid * NPT, NPT)],
                    den_hbm.at[cid, 1, pl.ds(sid * NPT, NPT)])


def _sc_scatter(wv, ex0, ex1, dst):
    return pl.kernel(
        _sc_scatter_body,
        out_type=[
            jax.ShapeDtypeStruct((NC, NPAD, HC), jnp.float32),
            jax.ShapeDtypeStruct((NC, 2, NPAD), jnp.float32),
        ],
        mesh=plsc.VectorSubcoreMesh(core_axis_name="c", subcore_axis_name="s"),
        scratch_types=[
            pltpu.VMEM((1, GW), jnp.int32),
            pltpu.VMEM((GW, HC), jnp.float32),
            pltpu.VMEM((GW,), jnp.float32),
            pltpu.VMEM((GW,), jnp.float32),
            pltpu.VMEM((1, GW), jnp.int32),
            pltpu.VMEM((GW, HC), jnp.float32),
            pltpu.VMEM((GW,), jnp.float32),
            pltpu.VMEM((GW,), jnp.float32),
            pltpu.VMEM((64, HC), jnp.float32),
            pltpu.VMEM((NPT,), jnp.float32),
            pltpu.VMEM_SHARED((NPAD, HC), jnp.float32),
            pltpu.VMEM_SHARED((NPAD,), jnp.float32),
            pltpu.VMEM_SHARED((NPAD,), jnp.float32),
        ] + [pltpu.SemaphoreType.DMA] * 8,
    )(wv, ex0, ex1, dst)


def kernel(x, edge_index, edge_time, msg, time_w, time_b, Wq, bq, Wk, bk, Wv, bv, We, Wskip, bskip):
    Wall = jnp.concatenate([Wq.T, Wk.T, Wv.T, Wskip.T], axis=1)
    ball = jnp.concatenate([bq, bk, bv, bskip])[None, :]

    q, kvp, skip = pl.pallas_call(
        _qkvg_body,
        grid=(N // 1000,),
        in_specs=[
            pl.BlockSpec((1000, D), lambda i: (i, 0)),
            pl.BlockSpec((D, 4 * HC), lambda i: (0, 0)),
            pl.BlockSpec((1, 4 * HC), lambda i: (0, 0)),
        ],
        out_specs=[
            pl.BlockSpec((1000, HC), lambda i: (i, 0)),
            pl.BlockSpec((1000, HC), lambda i: (i, 0)),
            pl.BlockSpec((1000, HC), lambda i: (i, 0)),
        ],
        out_shape=[
            jax.ShapeDtypeStruct((N, HC), jnp.float32),
            jax.ShapeDtypeStruct((N, HC), jnp.float32),
            jax.ShapeDtypeStruct((N, HC), jnp.float32),
        ],
    )(x, Wall, ball)

    wt = jnp.pad(time_w, (0, HC - TD))[:, None]
    bt = jnp.pad(time_b, (0, HC - TD))[:, None]
    WeTp = jnp.pad(We.T, ((0, HC - TD), (0, 0))).astype(jnp.bfloat16)
    t2 = edge_time[None, :]

    e = pl.pallas_call(
        _time_e_body,
        grid=(E // EBLK,),
        in_specs=[
            pl.BlockSpec((1, EBLK), lambda i: (0, i)),
            pl.BlockSpec((HC, 1), lambda i: (0, 0)),
            pl.BlockSpec((HC, 1), lambda i: (0, 0)),
            pl.BlockSpec((HC, HC), lambda i: (0, 0)),
        ],
        out_specs=pl.BlockSpec((EBLK, HC), lambda i: (i, 0)),
        out_shape=jax.ShapeDtypeStruct((E, HC), jnp.float32),
    )(t2, wt, bt, WeTp)

    src = edge_index[0]
    dst = edge_index[1]
    qd, kvs = _sc_gather(q, kvp, src, dst)

    ex0, ex1, wv = pl.pallas_call(
        _alpha_body,
        grid=(E // EBLK,),
        in_specs=[
            pl.BlockSpec((EBLK, HC), lambda i: (i, 0)),
            pl.BlockSpec((EBLK, HC), lambda i: (i, 0)),
            pl.BlockSpec((EBLK, HC), lambda i: (i, 0)),
        ],
        out_specs=[
            pl.BlockSpec((1, EBLK), lambda i: (0, i)),
            pl.BlockSpec((1, EBLK), lambda i: (0, i)),
            pl.BlockSpec((EBLK, HC), lambda i: (i, 0)),
        ],
        out_shape=[
            jax.ShapeDtypeStruct((1, E), jnp.float32),
            jax.ShapeDtypeStruct((1, E), jnp.float32),
            jax.ShapeDtypeStruct((E, HC), jnp.float32),
        ],
    )(qd, kvs, e)

    part, den = _sc_scatter(wv, ex0.reshape(E), ex1.reshape(E), dst)

    skip_pad = jnp.pad(skip, ((0, NPAD - N), (0, 0)))
    out_pad = pl.pallas_call(
        _final_body,
        grid=(NPAD // NBLK,),
        in_specs=[
            pl.BlockSpec((NC, NBLK, HC), lambda i: (0, i, 0)),
            pl.BlockSpec((NC, 2, NBLK), lambda i: (0, 0, i)),
            pl.BlockSpec((NBLK, HC), lambda i: (i, 0)),
        ],
        out_specs=pl.BlockSpec((NBLK, HC), lambda i: (i, 0)),
        out_shape=jax.ShapeDtypeStruct((NPAD, HC), jnp.float32),
    )(part, den, skip_pad)
    return out_pad[:N]


# speedup vs baseline: 1.8787x; 1.0012x over previous
import jax
import jax.numpy as jnp
from jax import lax
from jax.experimental import pallas as pl
from jax.experimental.pallas import tpu as pltpu
from jax.experimental.pallas import tpu_sc as plsc

N = 10000
E = 320000
D = 128
H = 2
C = 64
HC = H * C
TD = 100

NPAD = 10240
NBLK = 1024
EBLK = 6400

_SC_INFO = plsc.get_sparse_core_info()
NC = _SC_INFO.num_cores
NS = _SC_INFO.num_subcores
NW = NC * NS
EPW = E // NW
GW = 80
NWIN = EPW // GW
NPT = NPAD // NS


def _qkvg_body(x_ref, w_ref, b_ref, q_ref, kvp_ref, g_ref):
    o = (jnp.dot(x_ref[...], w_ref[...], preferred_element_type=jnp.float32)
         + b_ref[...])
    u32 = jnp.uint32
    kb = lax.bitcast_convert_type(
        o[:, HC:2 * HC].astype(jnp.bfloat16).astype(jnp.float32), u32)
    vb = lax.bitcast_convert_type(
        o[:, 2 * HC:3 * HC].astype(jnp.bfloat16).astype(jnp.float32), u32)
    kvp = (kb & jnp.uint32(0xFFFF0000)) | (vb >> 16)
    q_ref[...] = o[:, :HC]
    kvp_ref[...] = lax.bitcast_convert_type(kvp, jnp.float32)
    g_ref[...] = o[:, 3 * HC:]


def _time_e_body(t_ref, wt_ref, bt_ref, we_ref, e_ref):
    encT = jnp.cos(wt_ref[...] * t_ref[...] + bt_ref[...])
    enc = jnp.transpose(encT.astype(jnp.bfloat16))
    e_ref[...] = jnp.dot(enc, we_ref[...],
                         preferred_element_type=jnp.float32)


def _unpack(u, hi):
    h = (u & jnp.uint32(0xFFFF0000)) if hi else (u << 16)
    return lax.bitcast_convert_type(h, jnp.float32)


def _alpha_body(qd_ref, kvs_ref, e_ref, ex0_ref, ex1_ref, wv_ref):
    eb = e_ref[...]
    kv_u = lax.bitcast_convert_type(kvs_ref[...], jnp.uint32)
    ks = _unpack(kv_u, True)
    vs = _unpack(kv_u, False)
    a = qd_ref[...] * (ks + eb)
    h0 = jnp.sum(a[:, :C], axis=1) * (1.0 / 8.0)
    h1 = jnp.sum(a[:, C:], axis=1) * (1.0 / 8.0)
    ex0 = jnp.exp(h0)
    ex1 = jnp.exp(h1)
    ex0_ref[...] = ex0[None, :]
    ex1_ref[...] = ex1[None, :]
    w = jnp.concatenate(
        [jnp.broadcast_to(ex0[:, None], (EBLK, C)),
         jnp.broadcast_to(ex1[:, None], (EBLK, C))], axis=1)
    wv_ref[...] = (vs + eb) * w


def _final_body(p_ref, den_ref, skip_ref, o_ref):
    acc = p_ref[0] + p_ref[1]
    d0 = den_ref[0, 0] + den_ref[1, 0]
    d1 = den_ref[0, 1] + den_ref[1, 1]
    r0 = 1.0 / (d0 + 1e-16)
    r1 = 1.0 / (d1 + 1e-16)
    r = jnp.concatenate(
        [jnp.broadcast_to(r0[:, None], (NBLK, C)),
         jnp.broadcast_to(r1[:, None], (NBLK, C))], axis=1)
    o_ref[...] = acc * r + skip_ref[...]


def _sc_gather_body(q_hbm, kv_hbm, src_hbm, dst_hbm, qd_hbm, kvs_hbm,
                    idxs_v, idxd_v, qrows_v, kvrows_v, sem1, sem2):
    wid = lax.axis_index("s") * NC + lax.axis_index("c")
    base = wid * EPW

    def win(g, carry):
        b = pl.multiple_of(base + g * GW, 8)
        pltpu.sync_copy(src_hbm.at[pl.ds(b, GW)], idxs_v)
        pltpu.sync_copy(dst_hbm.at[pl.ds(b, GW)], idxd_v)
        c1 = pltpu.async_copy(q_hbm.at[idxd_v], qrows_v, sem1)
        c2 = pltpu.async_copy(kv_hbm.at[idxs_v], kvrows_v, sem2)
        c1.wait()
        c2.wait()
        pltpu.sync_copy(qrows_v, qd_hbm.at[pl.ds(b, GW)])
        pltpu.sync_copy(kvrows_v, kvs_hbm.at[pl.ds(b, GW)])
        return carry

    lax.fori_loop(0, NWIN, win, 0)


def _sc_gather(q, kvp, src, dst):
    return pl.kernel(
        _sc_gather_body,
        out_type=[
            jax.ShapeDtypeStruct((E, HC), jnp.float32),
            jax.ShapeDtypeStruct((E, HC), jnp.float32),
        ],
        mesh=plsc.VectorSubcoreMesh(core_axis_name="c", subcore_axis_name="s"),
        scratch_types=[
            pltpu.VMEM((GW,), jnp.int32),
            pltpu.VMEM((GW,), jnp.int32),
            pltpu.VMEM((GW, HC), jnp.float32),
            pltpu.VMEM((GW, HC), jnp.float32),
            pltpu.SemaphoreType.DMA,
            pltpu.SemaphoreType.DMA,
        ],
    )(q, kvp, src, dst)


def _sc_scatter_body(wv_hbm, ex0_hbm, ex1_hbm, dst_hbm,
                     part_hbm, den_hbm,
                     idxd_v, wrows_v, ex0_v, ex1_v,
                     idxd2_v, wrows2_v, ex02_v, ex12_v,
                     zr_v, zd_v, out_sh, den0_sh, den1_sh,
                     smi0, smw0, sm00, sm10, smi1, smw1, sm01, sm11):
    cid = lax.axis_index("c")
    sid = lax.axis_index("s")
    wid = sid * NC + cid
    base = wid * EPW

    zv16 = jnp.zeros((16,), jnp.float32)
    for r in range(zr_v.shape[0]):
        for cc in range(HC // 16):
            zr_v[r, pl.ds(cc * 16, 16)] = zv16
    for cc in range(NPT // 16):
        zd_v[pl.ds(cc * 16, 16)] = zv16
    nzr = zr_v.shape[0]
    for rep in range(NPT // nzr):
        pltpu.sync_copy(zr_v, out_sh.at[pl.ds(sid * NPT + rep * nzr, nzr)])
    pltpu.sync_copy(zd_v, den0_sh.at[pl.ds(sid * NPT, NPT)])
    pltpu.sync_copy(zd_v, den1_sh.at[pl.ds(sid * NPT, NPT)])
    plsc.subcore_barrier()

    def load(g, idxd, wr, e0, e1, si, sw, s0, s1):
        b = pl.multiple_of(base + g * GW, 8)
        pltpu.async_copy(dst_hbm.at[pl.ds(b, GW)], idxd.at[0], si)
        pltpu.async_copy(wv_hbm.at[pl.ds(b, GW)], wr, sw)
        pltpu.async_copy(ex0_hbm.at[pl.ds(b, GW)], e0, s0)
        pltpu.async_copy(ex1_hbm.at[pl.ds(b, GW)], e1, s1)

    def wait_l(g, idxd, wr, e0, e1, si, sw, s0, s1):
        b = pl.multiple_of(base + g * GW, 8)
        pltpu.make_async_copy(dst_hbm.at[pl.ds(b, GW)], idxd.at[0], si).wait()
        pltpu.make_async_copy(wv_hbm.at[pl.ds(b, GW)], wr, sw).wait()
        pltpu.make_async_copy(ex0_hbm.at[pl.ds(b, GW)], e0, s0).wait()
        pltpu.make_async_copy(ex1_hbm.at[pl.ds(b, GW)], e1, s1).wait()

    def scat(idxd, wr, e0, e1):
        pltpu.sync_copy(wr, out_sh.at[idxd.at[0]], add=True)
        pltpu.sync_copy(e0, den0_sh.at[idxd.at[0]], add=True)
        pltpu.sync_copy(e1, den1_sh.at[idxd.at[0]], add=True)

    set0 = (idxd_v, wrows_v, ex0_v, ex1_v, smi0, smw0, sm00, sm10)
    set1 = (idxd2_v, wrows2_v, ex02_v, ex12_v, smi1, smw1, sm01, sm11)

    load(0, *set0)

    def pair(h, carry):
        g0 = 2 * h
        g1 = g0 + 1
        wait_l(g0, *set0)
        load(g1, *set1)
        scat(idxd_v, wrows_v, ex0_v, ex1_v)
        wait_l(g1, *set1)

        @pl.when(g0 + 2 < NWIN)
        def _():
            load(g0 + 2, *set0)

        scat(idxd2_v, wrows2_v, ex02_v, ex12_v)
        return carry

    lax.fori_loop(0, NWIN // 2, pair, 0)
    wait_l(NWIN - 1, *set0)
    scat(idxd_v, wrows_v, ex0_v, ex1_v)
    plsc.subcore_barrier()

    pltpu.sync_copy(out_sh.at[pl.ds(sid * NPT, NPT)],
                    part_hbm.at[cid, pl.ds(sid * NPT, NPT)])
    pltpu.sync_copy(den0_sh.at[pl.ds(sid * NPT, NPT)],
                    den_hbm.at[cid, 0, pl.ds(sid * NPT, NPT)])
    pltpu.sync_copy(den1_sh.at[pl.ds(sid * NPT, NPT)],
                    den_hbm.at[cid, 1, pl.ds(sid * NPT, NPT)])


def _sc_scatter(wv, ex0, ex1, dst):
    return pl.kernel(
        _sc_scatter_body,
        out_type=[
            jax.ShapeDtypeStruct((NC, NPAD, HC), jnp.float32),
            jax.ShapeDtypeStruct((NC, 2, NPAD), jnp.float32),
        ],
        mesh=plsc.VectorSubcoreMesh(core_axis_name="c", subcore_axis_name="s"),
        scratch_types=[
            pltpu.VMEM((1, GW), jnp.int32),
            pltpu.VMEM((GW, HC), jnp.float32),
            pltpu.VMEM((GW,), jnp.float32),
            pltpu.VMEM((GW,), jnp.float32),
            pltpu.VMEM((1, GW), jnp.int32),
            pltpu.VMEM((GW, HC), jnp.float32),
            pltpu.VMEM((GW,), jnp.float32),
            pltpu.VMEM((GW,), jnp.float32),
            pltpu.VMEM((64, HC), jnp.float32),
            pltpu.VMEM((NPT,), jnp.float32),
            pltpu.VMEM_SHARED((NPAD, HC), jnp.float32),
            pltpu.VMEM_SHARED((NPAD,), jnp.float32),
            pltpu.VMEM_SHARED((NPAD,), jnp.float32),
        ] + [pltpu.SemaphoreType.DMA] * 8,
    )(wv, ex0, ex1, dst)


def kernel(x, edge_index, edge_time, msg, time_w, time_b, Wq, bq, Wk, bk, Wv, bv, We, Wskip, bskip):
    Wall = jnp.concatenate([Wq.T, Wk.T, Wv.T, Wskip.T], axis=1)
    ball = jnp.concatenate([bq, bk, bv, bskip])[None, :]

    q, kvp, skip = pl.pallas_call(
        _qkvg_body,
        grid=(N // 1000,),
        in_specs=[
            pl.BlockSpec((1000, D), lambda i: (i, 0)),
            pl.BlockSpec((D, 4 * HC), lambda i: (0, 0)),
            pl.BlockSpec((1, 4 * HC), lambda i: (0, 0)),
        ],
        out_specs=[
            pl.BlockSpec((1000, HC), lambda i: (i, 0)),
            pl.BlockSpec((1000, HC), lambda i: (i, 0)),
            pl.BlockSpec((1000, HC), lambda i: (i, 0)),
        ],
        out_shape=[
            jax.ShapeDtypeStruct((N, HC), jnp.float32),
            jax.ShapeDtypeStruct((N, HC), jnp.float32),
            jax.ShapeDtypeStruct((N, HC), jnp.float32),
        ],
    )(x, Wall, ball)

    wt = jnp.pad(time_w, (0, HC - TD))[:, None]
    bt = jnp.pad(time_b, (0, HC - TD))[:, None]
    WeTp = jnp.pad(We.T, ((0, HC - TD), (0, 0))).astype(jnp.bfloat16)
    t2 = edge_time[None, :]

    e = pl.pallas_call(
        _time_e_body,
        grid=(E // EBLK,),
        in_specs=[
            pl.BlockSpec((1, EBLK), lambda i: (0, i)),
            pl.BlockSpec((HC, 1), lambda i: (0, 0)),
            pl.BlockSpec((HC, 1), lambda i: (0, 0)),
            pl.BlockSpec((HC, HC), lambda i: (0, 0)),
        ],
        out_specs=pl.BlockSpec((EBLK, HC), lambda i: (i, 0)),
        out_shape=jax.ShapeDtypeStruct((E, HC), jnp.float32),
    )(t2, wt, bt, WeTp)

    src = edge_index[0]
    dst = edge_index[1]
    qd, kvs = _sc_gather(q, kvp, src, dst)

    ex0, ex1, wv = pl.pallas_call(
        _alpha_body,
        grid=(E // EBLK,),
        in_specs=[
            pl.BlockSpec((EBLK, HC), lambda i: (i, 0)),
            pl.BlockSpec((EBLK, HC), lambda i: (i, 0)),
            pl.BlockSpec((EBLK, HC), lambda i: (i, 0)),
        ],
        out_specs=[
            pl.BlockSpec((1, EBLK), lambda i: (0, i)),
            pl.BlockSpec((1, EBLK), lambda i: (0, i)),
            pl.BlockSpec((EBLK, HC), lambda i: (i, 0)),
        ],
        out_shape=[
            jax.ShapeDtypeStruct((1, E), jnp.float32),
            jax.ShapeDtypeStruct((1, E), jnp.float32),
            jax.ShapeDtypeStruct((E, HC), jnp.float32),
        ],
    )(qd, kvs, e)

    part, den = _sc_scatter(wv, ex0.reshape(E), ex1.reshape(E), dst)

    skip_pad = jnp.pad(skip, ((0, NPAD - N), (0, 0)))
    out_pad = pl.pallas_call(
        _final_body,
        grid=(NPAD // NBLK,),
        in_specs=[
            pl.BlockSpec((NC, NBLK, HC), lambda i: (0, i, 0)),
            pl.BlockSpec((NC, 2, NBLK), lambda i: (0, 0, i)),
            pl.BlockSpec((NBLK, HC), lambda i: (i, 0)),
        ],
        out_specs=pl.BlockSpec((NBLK, HC), lambda i: (i, 0)),
        out_shape=jax.ShapeDtypeStruct((NPAD, HC), jnp.float32),
    )(part, den, skip_pad)
    return out_pad[:N]
